# baseline (speedup 1.0000x reference)
import functools
import jax
import jax.numpy as jnp
from jax import lax
from jax.experimental import pallas as pl
from jax.experimental.pallas import tpu as pltpu
from jax.experimental.pallas import tpu_sc as plsc

B = 4
N = 20000
C = 21
NEG = -1e9
THR = 0.3
IOU_THR = 0.5
K = 100
KP = 112
BLK = 160
NBLK = 125
NBLKP = 128
MBLK = 112
MTOT = 2352
LANES = 16


def _tc_body(rel_ref, anch_ref, sc_ref, boxes_ref, probs_ref, bmax_ref, barg_ref):
    at = jnp.transpose(anch_ref[...], (1, 0))
    rt = jnp.transpose(rel_ref[0], (1, 0))
    ay1 = at[0]
    ax1 = at[1]
    ay2 = at[2]
    ax2 = at[3]
    ycenter_a = (ay1 + ay2) / 2.0
    xcenter_a = (ax1 + ax2) / 2.0
    ha = ay2 - ay1
    wa = ax2 - ax1
    ty = rt[0] / 10.0
    tx = rt[1] / 10.0
    th = rt[2] / 5.0
    tw = rt[3] / 5.0
    h = jnp.exp(th) * ha
    w = jnp.exp(tw) * wa
    yc = ty * ha + ycenter_a
    xc = tx * wa + xcenter_a
    boxes_ref[0, 0] = jnp.clip(yc - h / 2.0, 0.0, 512.0)
    boxes_ref[0, 1] = jnp.clip(xc - w / 2.0, 0.0, 512.0)
    boxes_ref[0, 2] = jnp.clip(yc + h / 2.0, 0.0, 512.0)
    boxes_ref[0, 3] = jnp.clip(xc + w / 2.0, 0.0, 512.0)
    p = 1.0 / (1.0 + jnp.exp(-sc_ref[0]))
    pt = jnp.where(p > THR, p, NEG)
    probs_ref[0] = jnp.transpose(pt, (1, 0))
    pt3 = pt.reshape(NBLK, BLK, C)
    m = jnp.max(pt3, axis=1)
    lane = lax.broadcasted_iota(jnp.int32, (NBLK, BLK, C), 1)
    il = jnp.min(jnp.where(pt3 == m[:, None, :], lane, _BIG_I), axis=1)
    row = lax.broadcasted_iota(jnp.int32, (NBLK, C), 0)
    flat = il + row * BLK
    mt = jnp.transpose(m, (1, 0))
    ft = jnp.transpose(flat, (1, 0))
    padf = jnp.full((C, NBLKP - NBLK), NEG, jnp.float32)
    padi = jnp.zeros((C, NBLKP + 16 - NBLK), jnp.int32)
    bmax_ref[0] = jnp.concatenate([mt, padf], axis=-1)
    barg_ref[0] = jnp.concatenate([ft, padi], axis=-1)


def _tc_stage(rel_codes, anchors, scores):
    return pl.pallas_call(
        _tc_body,
        grid=(B,),
        in_specs=[
            pl.BlockSpec((1, N, 4), lambda b: (b, 0, 0)),
            pl.BlockSpec((N, 4), lambda b: (0, 0)),
            pl.BlockSpec((1, N, C), lambda b: (b, 0, 0)),
        ],
        out_specs=[
            pl.BlockSpec((1, 4, N), lambda b: (b, 0, 0)),
            pl.BlockSpec((1, C, N), lambda b: (b, 0, 0)),
            pl.BlockSpec((1, C, NBLKP), lambda b: (b, 0, 0)),
            pl.BlockSpec((1, C, NBLKP + 16), lambda b: (b, 0, 0)),
        ],
        out_shape=[
            jax.ShapeDtypeStruct((B, 4, N), jnp.float32),
            jax.ShapeDtypeStruct((B, C, N), jnp.float32),
            jax.ShapeDtypeStruct((B, C, NBLKP), jnp.float32),
            jax.ShapeDtypeStruct((B, C, NBLKP + 16), jnp.int32),
        ],
    )(rel_codes, anchors, scores)


_IOTA = functools.partial(lax.iota, jnp.int32, LANES)
_BIG_I = 1 << 30


def _vex_f(ref, idx):
    base = (idx // LANES) * LANES
    v = ref[pl.ds(base, LANES)]
    return lax.reduce_sum(jnp.where(_IOTA() == idx - base, v, 0.0), (0,))


def _vex_i(ref, idx):
    base = (idx // LANES) * LANES
    v = ref[pl.ds(base, LANES)]
    return lax.reduce_sum(jnp.where(_IOTA() == idx - base, v, 0), (0,))


def _vbro(ref, idx):
    base = (idx // LANES) * LANES
    v = ref[pl.ds(base, LANES)]
    lanes = jnp.full((LANES,), idx - base, jnp.int32)
    return v.at[lanes].get(mode="promise_in_bounds")


def _vset(ref, idx, val):
    base = (idx // LANES) * LANES
    v = ref[pl.ds(base, LANES)]
    ref[pl.ds(base, LANES)] = jnp.where(_IOTA() == idx - base, val, v)


def _scan_range(ref, start, nslices, unroll=False):
    def step(i, carry):
        vmax, vidx = carry
        off = start + i * LANES
        v = ref[pl.ds(off, LANES)]
        take = v > vmax
        return (jnp.where(take, v, vmax),
                jnp.where(take, off + _IOTA(), vidx))
    carry = (jnp.full((LANES,), NEG, jnp.float32), jnp.zeros((LANES,), jnp.int32))
    if unroll:
        for i in range(nslices):
            carry = step(i, carry)
        vmax, vidx = carry
    else:
        vmax, vidx = lax.fori_loop(0, nslices, step, carry)
    m = lax.reduce_max(vmax, (0,))
    idx = lax.reduce_min(jnp.where(vmax == m, vidx, _BIG_I), (0,))
    return m, idx


def _find_global(bm, nslices):
    return _scan_range(bm, 0, nslices, unroll=True)


def _nms_class(pv, py1, px1, py2, px2, bm, ba, sy1, sx1, sy2, sx2, ssc):
    zeros16 = jnp.zeros((LANES,), jnp.float32)
    negs16 = jnp.full((LANES,), NEG, jnp.float32)

    def init_sel(i, _):
        off = pl.ds(i * LANES, LANES)
        sy1[off] = zeros16
        sx1[off] = zeros16
        sy2[off] = zeros16
        sx2[off] = zeros16
        ssc[off] = negs16
        return 0
    lax.fori_loop(0, KP // LANES, init_sel, 0)

    g0, gb0 = _find_global(bm, NBLKP // LANES)

    def nms_cond(carry):
        nsel, gmax, _ = carry
        return (nsel < K) & (gmax > 0.0)

    def nms_body(carry):
        nsel, gmax, gblk = carry
        idx = _vex_i(ba, gblk)
        by1 = _vbro(py1, idx)
        bx1 = _vbro(px1, idx)
        by2 = _vbro(py2, idx)
        bx2 = _vbro(px2, idx)
        a1 = (by2 - by1) * (bx2 - bx1)

        def iou_step(i, acc):
            off = pl.ds(i * LANES, LANES)
            vy1 = sy1[off]
            vx1 = sx1[off]
            vy2 = sy2[off]
            vx2 = sx2[off]
            yy1 = jnp.maximum(by1, vy1)
            xx1 = jnp.maximum(bx1, vx1)
            yy2 = jnp.minimum(by2, vy2)
            xx2 = jnp.minimum(bx2, vx2)
            inter = (jnp.maximum(yy2 - yy1, 0.0)
                     * jnp.maximum(xx2 - xx1, 0.0))
            a2 = (vy2 - vy1) * (vx2 - vx1)
            iou = inter / (a1 + a2 - inter + 1e-8)
            return acc | (iou > IOU_THR)
        supm = jnp.zeros((LANES,), jnp.bool_)
        for i in range(KP // LANES):
            supm = iou_step(i, supm)
        sup = jnp.any(supm)

        _vset(sy1, nsel, jnp.where(sup, 0.0, by1))
        _vset(sx1, nsel, jnp.where(sup, 0.0, bx1))
        _vset(sy2, nsel, jnp.where(sup, 0.0, by2))
        _vset(sx2, nsel, jnp.where(sup, 0.0, bx2))
        _vset(ssc, nsel, jnp.where(sup, NEG, gmax))
        nsel = nsel + jnp.where(sup, 0, 1)

        _vset(pv, idx, NEG)
        m, fidx = _scan_range(pv, gblk * BLK, BLK // LANES, unroll=True)
        _vset(bm, gblk, m)
        _vset(ba, gblk, fidx)
        gmax2, gblk2 = _find_global(bm, NBLKP // LANES)
        return nsel, gmax2, gblk2

    lax.while_loop(nms_cond, nms_body, (jnp.int32(0), g0, gb0))


def _merge_batch(mc_y1, mc_x1, mc_y2, mc_x2, mc_sc, mbm, mba,
                 st_bx, st_sc, st_lb, st_nv):
    zeros16 = jnp.zeros((LANES,), jnp.float32)

    def init_mblk(b, _):
        m, idx = _scan_range(mc_sc, b * MBLK, MBLK // LANES)
        _vset(mbm, b, m)
        _vset(mba, b, idx)
        return 0
    lax.fori_loop(0, MTOT // MBLK, init_mblk, 0)

    def pad_mblk(i, _):
        _vset(mbm, MTOT // MBLK + i, NEG)
        _vset(mba, MTOT // MBLK + i, 0)
        return 0
    lax.fori_loop(0, 2 * LANES - MTOT // MBLK, pad_mblk, 0)

    def sel_step(i, nv):
        s, blk = _find_global(mbm, 2)
        f = _vex_i(mba, blk)
        valid = s > NEG / 2.0
        cc = f // KP
        by1 = _vbro(mc_y1, f)
        bx1 = _vbro(mc_x1, f)
        by2 = _vbro(mc_y2, f)
        bx2 = _vbro(mc_x2, f)
        sw = jnp.where(valid, s, 0.0)
        lw = jnp.where(valid, cc.astype(jnp.float32), 0.0)
        _vset(st_sc, i, sw)
        _vset(st_lb, i, lw)
        base = (i * 4 // LANES) * LANES
        off = i * 4 - base
        io = _IOTA()
        v = st_bx[pl.ds(base, LANES)]
        v = jnp.where(io == off, jnp.where(valid, by1, 0.0), v)
        v = jnp.where(io == off + 1, jnp.where(valid, bx1, 0.0), v)
        v = jnp.where(io == off + 2, jnp.where(valid, by2, 0.0), v)
        v = jnp.where(io == off + 3, jnp.where(valid, bx2, 0.0), v)
        st_bx[pl.ds(base, LANES)] = v
        _vset(mc_sc, f, NEG)
        m, fidx = _scan_range(mc_sc, blk * MBLK, MBLK // LANES, unroll=True)
        _vset(mbm, blk, m)
        _vset(mba, blk, fidx)
        return nv + jnp.where(valid, 1, 0)
    nv = lax.fori_loop(0, K, sel_step, jnp.int32(0))

    def pad_out(i, _):
        _vset(st_sc, K + i, 0.0)
        _vset(st_lb, K + i, 0.0)
        return 0
    lax.fori_loop(0, KP - K, pad_out, 0)

    def pad_bx(i, _):
        st_bx[pl.ds(K * 4 + i * LANES, LANES)] = zeros16
        return 0
    lax.fori_loop(0, (KP - K) * 4 // LANES, pad_bx, 0)

    st_nv[pl.ds(0, LANES)] = jnp.where(_IOTA() == 0, nv, 0)


def _sc_body(probs_hbm, boxes_hbm, bmax_hbm, barg_hbm, oboxes, oscores, olabels, onv,
             py1, px1, py2, px2, pv, bm, ba,
             sy1, sx1, sy2, sx2, ssc,
             sh_y1, sh_x1, sh_y2, sh_x2, sh_sc,
             mc_y1, mc_x1, mc_y2, mc_x2, mc_sc,
             mbm, mba, st_bx, st_sc, st_lb, st_nv):
    cidx = lax.axis_index("c")
    sidx = lax.axis_index("s")
    batch = cidx * 2 + sidx // 8
    j = sidx % 8
    bb = sidx // 8

    pltpu.sync_copy(boxes_hbm.at[batch, 0], py1)
    pltpu.sync_copy(boxes_hbm.at[batch, 1], px1)
    pltpu.sync_copy(boxes_hbm.at[batch, 2], py2)
    pltpu.sync_copy(boxes_hbm.at[batch, 3], px2)

    zeros16 = jnp.zeros((LANES,), jnp.float32)
    negs16 = jnp.full((LANES,), NEG, jnp.float32)

    for t in range(3):
        c = j + 8 * t

        @pl.when(c < C)
        def _():
            pltpu.sync_copy(probs_hbm.at[batch, c], pv)
            pltpu.sync_copy(bmax_hbm.at[batch, c], bm)
            pltpu.sync_copy(barg_hbm.at[batch, c], ba)
            _nms_class(pv, py1, px1, py2, px2, bm, ba, sy1, sx1, sy2, sx2, ssc)
            sh_off = bb * (C * KP + 16) + c * KP
            pltpu.sync_copy(sy1, sh_y1.at[pl.ds(sh_off, KP)])
            pltpu.sync_copy(sx1, sh_x1.at[pl.ds(sh_off, KP)])
            pltpu.sync_copy(sy2, sh_y2.at[pl.ds(sh_off, KP)])
            pltpu.sync_copy(sx2, sh_x2.at[pl.ds(sh_off, KP)])
            pltpu.sync_copy(ssc, sh_sc.at[pl.ds(sh_off, KP)])

    plsc.subcore_barrier()

    @pl.when(j == 5)
    def _():
        sh_b = bb * (C * KP + 16)
        pltpu.sync_copy(sh_y1.at[pl.ds(sh_b, C * KP + 16)], mc_y1)
        pltpu.sync_copy(sh_x1.at[pl.ds(sh_b, C * KP + 16)], mc_x1)
        pltpu.sync_copy(sh_y2.at[pl.ds(sh_b, C * KP + 16)], mc_y2)
        pltpu.sync_copy(sh_x2.at[pl.ds(sh_b, C * KP + 16)], mc_x2)
        pltpu.sync_copy(sh_sc.at[pl.ds(sh_b, C * KP)], mc_sc)

        _merge_batch(mc_y1, mc_x1, mc_y2, mc_x2, mc_sc, mbm, mba,
                     st_bx, st_sc, st_lb, st_nv)

        pltpu.sync_copy(st_bx, oboxes.at[batch])
        pltpu.sync_copy(st_sc, oscores.at[batch])
        pltpu.sync_copy(st_lb, olabels.at[batch])
        pltpu.sync_copy(st_nv, onv.at[batch])


def _sc_stage(probs, boxes_t, bmax, barg):
    mesh = plsc.VectorSubcoreMesh(core_axis_name="c", subcore_axis_name="s")
    f = pl.kernel(
        _sc_body,
        out_type=[
            jax.ShapeDtypeStruct((B, KP * 4), jnp.float32),
            jax.ShapeDtypeStruct((B, KP), jnp.float32),
            jax.ShapeDtypeStruct((B, KP), jnp.float32),
            jax.ShapeDtypeStruct((B, LANES), jnp.int32),
        ],
        mesh=mesh,
        compiler_params=pltpu.CompilerParams(needs_layout_passes=False),
        scratch_types=[
            pltpu.VMEM((N + 16,), jnp.float32),
            pltpu.VMEM((N + 16,), jnp.float32),
            pltpu.VMEM((N + 16,), jnp.float32),
            pltpu.VMEM((N + 16,), jnp.float32),
            pltpu.VMEM((N,), jnp.float32),
            pltpu.VMEM((NBLKP,), jnp.float32),
            pltpu.VMEM((NBLKP + 16,), jnp.int32),
            pltpu.VMEM((KP,), jnp.float32),
            pltpu.VMEM((KP,), jnp.float32),
            pltpu.VMEM((KP,), jnp.float32),
            pltpu.VMEM((KP,), jnp.float32),
            pltpu.VMEM((KP,), jnp.float32),
            pltpu.VMEM_SHARED((2 * (C * KP + 16),), jnp.float32),
            pltpu.VMEM_SHARED((2 * (C * KP + 16),), jnp.float32),
            pltpu.VMEM_SHARED((2 * (C * KP + 16),), jnp.float32),
            pltpu.VMEM_SHARED((2 * (C * KP + 16),), jnp.float32),
            pltpu.VMEM_SHARED((2 * (C * KP + 16),), jnp.float32),
            pltpu.VMEM((MTOT + 16,), jnp.float32),
            pltpu.VMEM((MTOT + 16,), jnp.float32),
            pltpu.VMEM((MTOT + 16,), jnp.float32),
            pltpu.VMEM((MTOT + 16,), jnp.float32),
            pltpu.VMEM((MTOT,), jnp.float32),
            pltpu.VMEM((2 * LANES,), jnp.float32),
            pltpu.VMEM((3 * LANES,), jnp.int32),
            pltpu.VMEM((KP * 4,), jnp.float32),
            pltpu.VMEM((KP,), jnp.float32),
            pltpu.VMEM((KP,), jnp.float32),
            pltpu.VMEM((LANES,), jnp.int32),
        ],
    )
    return f(probs, boxes_t, bmax, barg)


def kernel(rel_codes, scores, anchors):
    boxes3d, probs, bmax, barg = _tc_stage(rel_codes, anchors, scores)
    boxes_t = jnp.pad(boxes3d, ((0, 0), (0, 0), (0, 16)))
    bx, sc, lb, nv = _sc_stage(probs, boxes_t, bmax, barg)
    out_boxes = bx.reshape(B, KP, 4)[:, :K, :]
    out_scores = sc[:, :K]
    out_labels = lb[:, :K]
    num_valid = nv[:, 0]
    return out_boxes, out_scores, out_labels, num_valid


# speedup vs baseline: 1.3333x; 1.3333x over previous
import functools
import jax
import jax.numpy as jnp
from jax import lax
from jax.experimental import pallas as pl
from jax.experimental.pallas import tpu as pltpu
from jax.experimental.pallas import tpu_sc as plsc

B = 4
N = 20000
C = 21
NEG = -1e9
THR = 0.3
IOU_THR = 0.5
K = 100
KP = 112
BLK = 160
NBLK = 125
NBLKP = 128
MBLK = 112
MTOT = 2352
LANES = 16


def _tc_body(rel_ref, anch_ref, sc_ref, boxes_ref, probs_ref, bmax_ref, barg_ref):
    ay1 = anch_ref[0]
    ax1 = anch_ref[1]
    ay2 = anch_ref[2]
    ax2 = anch_ref[3]
    ycenter_a = (ay1 + ay2) / 2.0
    xcenter_a = (ax1 + ax2) / 2.0
    ha = ay2 - ay1
    wa = ax2 - ax1
    for b in range(B):
        ty = rel_ref[b, 0] / 10.0
        tx = rel_ref[b, 1] / 10.0
        th = rel_ref[b, 2] / 5.0
        tw = rel_ref[b, 3] / 5.0
        h = jnp.exp(th) * ha
        w = jnp.exp(tw) * wa
        yc = ty * ha + ycenter_a
        xc = tx * wa + xcenter_a
        boxes_ref[b, 0] = jnp.clip(yc - h / 2.0, 0.0, 512.0)
        boxes_ref[b, 1] = jnp.clip(xc - w / 2.0, 0.0, 512.0)
        boxes_ref[b, 2] = jnp.clip(yc + h / 2.0, 0.0, 512.0)
        boxes_ref[b, 3] = jnp.clip(xc + w / 2.0, 0.0, 512.0)
        p = 1.0 / (1.0 + jnp.exp(-sc_ref[b]))
        pt = jnp.where(p > THR, p, NEG)
        probs_ref[b] = pt
        m = jnp.max(pt, axis=-1)
        lane = lax.broadcasted_iota(jnp.int32, (C, NBLK, BLK), 2)
        il = jnp.min(jnp.where(pt == m[..., None], lane, _BIG_I), axis=-1)
        row = lax.broadcasted_iota(jnp.int32, (C, NBLK), 1)
        flat = il + row * BLK
        padf = jnp.full((C, NBLKP - NBLK), NEG, jnp.float32)
        padi = jnp.zeros((C, NBLKP + 16 - NBLK), jnp.int32)
        bmax_ref[b] = jnp.concatenate([m, padf], axis=-1)
        barg_ref[b] = jnp.concatenate([flat, padi], axis=-1)


def _tc_stage(rel_t, anch_t, scores_t):
    return pl.pallas_call(
        _tc_body,
        out_shape=[
            jax.ShapeDtypeStruct((B, 4, NBLK, BLK), jnp.float32),
            jax.ShapeDtypeStruct((B, C, NBLK, BLK), jnp.float32),
            jax.ShapeDtypeStruct((B, C, NBLKP), jnp.float32),
            jax.ShapeDtypeStruct((B, C, NBLKP + 16), jnp.int32),
        ],
    )(rel_t, anch_t, scores_t)


_IOTA = functools.partial(lax.iota, jnp.int32, LANES)
_BIG_I = 1 << 30


def _vex_f(ref, idx):
    base = (idx // LANES) * LANES
    v = ref[pl.ds(base, LANES)]
    return lax.reduce_sum(jnp.where(_IOTA() == idx - base, v, 0.0), (0,))


def _vex_i(ref, idx):
    base = (idx // LANES) * LANES
    v = ref[pl.ds(base, LANES)]
    return lax.reduce_sum(jnp.where(_IOTA() == idx - base, v, 0), (0,))


def _vbro(ref, idx):
    base = (idx // LANES) * LANES
    v = ref[pl.ds(base, LANES)]
    lanes = jnp.full((LANES,), idx - base, jnp.int32)
    return v.at[lanes].get(mode="promise_in_bounds")


def _vset(ref, idx, val):
    base = (idx // LANES) * LANES
    v = ref[pl.ds(base, LANES)]
    ref[pl.ds(base, LANES)] = jnp.where(_IOTA() == idx - base, val, v)


def _scan_range(ref, start, nslices, unroll=False):
    def step(i, carry):
        vmax, vidx = carry
        off = start + i * LANES
        v = ref[pl.ds(off, LANES)]
        take = v > vmax
        return (jnp.where(take, v, vmax),
                jnp.where(take, off + _IOTA(), vidx))
    carry = (jnp.full((LANES,), NEG, jnp.float32), jnp.zeros((LANES,), jnp.int32))
    if unroll:
        for i in range(nslices):
            carry = step(i, carry)
        vmax, vidx = carry
    else:
        vmax, vidx = lax.fori_loop(0, nslices, step, carry)
    m = lax.reduce_max(vmax, (0,))
    idx = lax.reduce_min(jnp.where(vmax == m, vidx, _BIG_I), (0,))
    return m, idx


def _find_global(bm, nslices):
    return _scan_range(bm, 0, nslices, unroll=True)


def _nms_class(pv, py1, px1, py2, px2, bm, ba, sy1, sx1, sy2, sx2, ssc):
    zeros16 = jnp.zeros((LANES,), jnp.float32)
    negs16 = jnp.full((LANES,), NEG, jnp.float32)

    def init_sel(i, _):
        off = pl.ds(i * LANES, LANES)
        sy1[off] = zeros16
        sx1[off] = zeros16
        sy2[off] = zeros16
        sx2[off] = zeros16
        ssc[off] = negs16
        return 0
    lax.fori_loop(0, KP // LANES, init_sel, 0)

    g0, gb0 = _find_global(bm, NBLKP // LANES)

    def nms_cond(carry):
        nsel, gmax, _ = carry
        return (nsel < K) & (gmax > 0.0)

    def nms_body(carry):
        nsel, gmax, gblk = carry
        idx = _vex_i(ba, gblk)
        by1 = _vbro(py1, idx)
        bx1 = _vbro(px1, idx)
        by2 = _vbro(py2, idx)
        bx2 = _vbro(px2, idx)
        a1 = (by2 - by1) * (bx2 - bx1)

        def iou_step(i, acc):
            off = pl.ds(i * LANES, LANES)
            vy1 = sy1[off]
            vx1 = sx1[off]
            vy2 = sy2[off]
            vx2 = sx2[off]
            yy1 = jnp.maximum(by1, vy1)
            xx1 = jnp.maximum(bx1, vx1)
            yy2 = jnp.minimum(by2, vy2)
            xx2 = jnp.minimum(bx2, vx2)
            inter = (jnp.maximum(yy2 - yy1, 0.0)
                     * jnp.maximum(xx2 - xx1, 0.0))
            a2 = (vy2 - vy1) * (vx2 - vx1)
            iou = inter / (a1 + a2 - inter + 1e-8)
            return acc | (iou > IOU_THR)
        supm = jnp.zeros((LANES,), jnp.bool_)
        for i in range(KP // LANES):
            supm = iou_step(i, supm)
        sup = jnp.any(supm)

        _vset(sy1, nsel, jnp.where(sup, 0.0, by1))
        _vset(sx1, nsel, jnp.where(sup, 0.0, bx1))
        _vset(sy2, nsel, jnp.where(sup, 0.0, by2))
        _vset(sx2, nsel, jnp.where(sup, 0.0, bx2))
        _vset(ssc, nsel, jnp.where(sup, NEG, gmax))
        nsel = nsel + jnp.where(sup, 0, 1)

        _vset(pv, idx, NEG)
        m, fidx = _scan_range(pv, gblk * BLK, BLK // LANES, unroll=True)
        _vset(bm, gblk, m)
        _vset(ba, gblk, fidx)
        gmax2, gblk2 = _find_global(bm, NBLKP // LANES)
        return nsel, gmax2, gblk2

    lax.while_loop(nms_cond, nms_body, (jnp.int32(0), g0, gb0))


def _merge_batch(mc_y1, mc_x1, mc_y2, mc_x2, mc_sc, mbm, mba,
                 st_bx, st_sc, st_lb, st_nv):
    zeros16 = jnp.zeros((LANES,), jnp.float32)

    def init_mblk(b, _):
        m, idx = _scan_range(mc_sc, b * MBLK, MBLK // LANES)
        _vset(mbm, b, m)
        _vset(mba, b, idx)
        return 0
    lax.fori_loop(0, MTOT // MBLK, init_mblk, 0)

    def pad_mblk(i, _):
        _vset(mbm, MTOT // MBLK + i, NEG)
        _vset(mba, MTOT // MBLK + i, 0)
        return 0
    lax.fori_loop(0, 2 * LANES - MTOT // MBLK, pad_mblk, 0)

    def sel_step(i, nv):
        s, blk = _find_global(mbm, 2)
        f = _vex_i(mba, blk)
        valid = s > NEG / 2.0
        cc = f // KP
        by1 = _vbro(mc_y1, f)
        bx1 = _vbro(mc_x1, f)
        by2 = _vbro(mc_y2, f)
        bx2 = _vbro(mc_x2, f)
        sw = jnp.where(valid, s, 0.0)
        lw = jnp.where(valid, cc.astype(jnp.float32), 0.0)
        _vset(st_sc, i, sw)
        _vset(st_lb, i, lw)
        base = (i * 4 // LANES) * LANES
        off = i * 4 - base
        io = _IOTA()
        v = st_bx[pl.ds(base, LANES)]
        v = jnp.where(io == off, jnp.where(valid, by1, 0.0), v)
        v = jnp.where(io == off + 1, jnp.where(valid, bx1, 0.0), v)
        v = jnp.where(io == off + 2, jnp.where(valid, by2, 0.0), v)
        v = jnp.where(io == off + 3, jnp.where(valid, bx2, 0.0), v)
        st_bx[pl.ds(base, LANES)] = v
        _vset(mc_sc, f, NEG)
        m, fidx = _scan_range(mc_sc, blk * MBLK, MBLK // LANES, unroll=True)
        _vset(mbm, blk, m)
        _vset(mba, blk, fidx)
        return nv + jnp.where(valid, 1, 0)
    nv = lax.fori_loop(0, K, sel_step, jnp.int32(0))

    def pad_out(i, _):
        _vset(st_sc, K + i, 0.0)
        _vset(st_lb, K + i, 0.0)
        return 0
    lax.fori_loop(0, KP - K, pad_out, 0)

    def pad_bx(i, _):
        st_bx[pl.ds(K * 4 + i * LANES, LANES)] = zeros16
        return 0
    lax.fori_loop(0, (KP - K) * 4 // LANES, pad_bx, 0)

    st_nv[pl.ds(0, LANES)] = jnp.where(_IOTA() == 0, nv, 0)


def _sc_body(probs_hbm, boxes_hbm, bmax_hbm, barg_hbm, oboxes, oscores, olabels, onv,
             py1, px1, py2, px2, pv, bm, ba,
             sy1, sx1, sy2, sx2, ssc,
             sh_y1, sh_x1, sh_y2, sh_x2, sh_sc,
             mc_y1, mc_x1, mc_y2, mc_x2, mc_sc,
             mbm, mba, st_bx, st_sc, st_lb, st_nv):
    cidx = lax.axis_index("c")
    sidx = lax.axis_index("s")
    batch = cidx * 2 + sidx // 8
    j = sidx % 8
    bb = sidx // 8

    pltpu.sync_copy(boxes_hbm.at[batch, 0], py1)
    pltpu.sync_copy(boxes_hbm.at[batch, 1], px1)
    pltpu.sync_copy(boxes_hbm.at[batch, 2], py2)
    pltpu.sync_copy(boxes_hbm.at[batch, 3], px2)

    zeros16 = jnp.zeros((LANES,), jnp.float32)
    negs16 = jnp.full((LANES,), NEG, jnp.float32)

    for t in range(3):
        c = j + 8 * t

        @pl.when(c < C)
        def _():
            pltpu.sync_copy(probs_hbm.at[batch, c], pv)
            pltpu.sync_copy(bmax_hbm.at[batch, c], bm)
            pltpu.sync_copy(barg_hbm.at[batch, c], ba)
            _nms_class(pv, py1, px1, py2, px2, bm, ba, sy1, sx1, sy2, sx2, ssc)
            sh_off = bb * (C * KP + 16) + c * KP
            pltpu.sync_copy(sy1, sh_y1.at[pl.ds(sh_off, KP)])
            pltpu.sync_copy(sx1, sh_x1.at[pl.ds(sh_off, KP)])
            pltpu.sync_copy(sy2, sh_y2.at[pl.ds(sh_off, KP)])
            pltpu.sync_copy(sx2, sh_x2.at[pl.ds(sh_off, KP)])
            pltpu.sync_copy(ssc, sh_sc.at[pl.ds(sh_off, KP)])

    plsc.subcore_barrier()

    @pl.when(j == 5)
    def _():
        sh_b = bb * (C * KP + 16)
        pltpu.sync_copy(sh_y1.at[pl.ds(sh_b, C * KP + 16)], mc_y1)
        pltpu.sync_copy(sh_x1.at[pl.ds(sh_b, C * KP + 16)], mc_x1)
        pltpu.sync_copy(sh_y2.at[pl.ds(sh_b, C * KP + 16)], mc_y2)
        pltpu.sync_copy(sh_x2.at[pl.ds(sh_b, C * KP + 16)], mc_x2)
        pltpu.sync_copy(sh_sc.at[pl.ds(sh_b, C * KP)], mc_sc)

        _merge_batch(mc_y1, mc_x1, mc_y2, mc_x2, mc_sc, mbm, mba,
                     st_bx, st_sc, st_lb, st_nv)

        pltpu.sync_copy(st_bx, oboxes.at[batch])
        pltpu.sync_copy(st_sc, oscores.at[batch])
        pltpu.sync_copy(st_lb, olabels.at[batch])
        pltpu.sync_copy(st_nv, onv.at[batch])


def _sc_stage(probs, boxes_t, bmax, barg):
    mesh = plsc.VectorSubcoreMesh(core_axis_name="c", subcore_axis_name="s")
    f = pl.kernel(
        _sc_body,
        out_type=[
            jax.ShapeDtypeStruct((B, KP * 4), jnp.float32),
            jax.ShapeDtypeStruct((B, KP), jnp.float32),
            jax.ShapeDtypeStruct((B, KP), jnp.float32),
            jax.ShapeDtypeStruct((B, LANES), jnp.int32),
        ],
        mesh=mesh,
        compiler_params=pltpu.CompilerParams(needs_layout_passes=False),
        scratch_types=[
            pltpu.VMEM((N + 16,), jnp.float32),
            pltpu.VMEM((N + 16,), jnp.float32),
            pltpu.VMEM((N + 16,), jnp.float32),
            pltpu.VMEM((N + 16,), jnp.float32),
            pltpu.VMEM((N,), jnp.float32),
            pltpu.VMEM((NBLKP,), jnp.float32),
            pltpu.VMEM((NBLKP + 16,), jnp.int32),
            pltpu.VMEM((KP,), jnp.float32),
            pltpu.VMEM((KP,), jnp.float32),
            pltpu.VMEM((KP,), jnp.float32),
            pltpu.VMEM((KP,), jnp.float32),
            pltpu.VMEM((KP,), jnp.float32),
            pltpu.VMEM_SHARED((2 * (C * KP + 16),), jnp.float32),
            pltpu.VMEM_SHARED((2 * (C * KP + 16),), jnp.float32),
            pltpu.VMEM_SHARED((2 * (C * KP + 16),), jnp.float32),
            pltpu.VMEM_SHARED((2 * (C * KP + 16),), jnp.float32),
            pltpu.VMEM_SHARED((2 * (C * KP + 16),), jnp.float32),
            pltpu.VMEM((MTOT + 16,), jnp.float32),
            pltpu.VMEM((MTOT + 16,), jnp.float32),
            pltpu.VMEM((MTOT + 16,), jnp.float32),
            pltpu.VMEM((MTOT + 16,), jnp.float32),
            pltpu.VMEM((MTOT,), jnp.float32),
            pltpu.VMEM((2 * LANES,), jnp.float32),
            pltpu.VMEM((3 * LANES,), jnp.int32),
            pltpu.VMEM((KP * 4,), jnp.float32),
            pltpu.VMEM((KP,), jnp.float32),
            pltpu.VMEM((KP,), jnp.float32),
            pltpu.VMEM((LANES,), jnp.int32),
        ],
    )
    return f(probs, boxes_t, bmax, barg)


def kernel(rel_codes, scores, anchors):
    rel_t = jnp.transpose(rel_codes, (0, 2, 1)).reshape(B, 4, NBLK, BLK)
    anch_t = jnp.transpose(anchors, (1, 0)).reshape(4, NBLK, BLK)
    scores_t = jnp.transpose(scores, (0, 2, 1)).reshape(B, C, NBLK, BLK)
    boxes4d, probs4d, bmax, barg = _tc_stage(rel_t, anch_t, scores_t)
    boxes_t = jnp.pad(boxes4d.reshape(B, 4, N), ((0, 0), (0, 0), (0, 16)))
    probs = probs4d.reshape(B, C, N)
    bx, sc, lb, nv = _sc_stage(probs, boxes_t, bmax, barg)
    out_boxes = bx.reshape(B, KP, 4)[:, :K, :]
    out_scores = sc[:, :K]
    out_labels = lb[:, :K]
    num_valid = nv[:, 0]
    return out_boxes, out_scores, out_labels, num_valid


# speedup vs baseline: 1.4045x; 1.0534x over previous
import functools
import jax
import jax.numpy as jnp
from jax import lax
from jax.experimental import pallas as pl
from jax.experimental.pallas import tpu as pltpu
from jax.experimental.pallas import tpu_sc as plsc

B = 4
N = 20000
C = 21
NEG = -1e9
THR = 0.3
IOU_THR = 0.5
K = 100
KP = 112
BLK = 160
NBLK = 125
NBLKP = 128
MBLK = 112
MTOT = 2352
LANES = 16


def _tc_body(rel_ref, anch_ref, sc_ref, boxes_ref, probs_ref, bmax_ref, barg_ref):
    ay1 = anch_ref[0]
    ax1 = anch_ref[1]
    ay2 = anch_ref[2]
    ax2 = anch_ref[3]
    ycenter_a = (ay1 + ay2) / 2.0
    xcenter_a = (ax1 + ax2) / 2.0
    ha = ay2 - ay1
    wa = ax2 - ax1
    ty = rel_ref[0] / 10.0
    tx = rel_ref[1] / 10.0
    th = rel_ref[2] / 5.0
    tw = rel_ref[3] / 5.0
    h = jnp.exp(th) * ha
    w = jnp.exp(tw) * wa
    yc = ty * ha + ycenter_a
    xc = tx * wa + xcenter_a
    boxes_ref[0] = jnp.clip(yc - h / 2.0, 0.0, 512.0)
    boxes_ref[1] = jnp.clip(xc - w / 2.0, 0.0, 512.0)
    boxes_ref[2] = jnp.clip(yc + h / 2.0, 0.0, 512.0)
    boxes_ref[3] = jnp.clip(xc + w / 2.0, 0.0, 512.0)
    p = 1.0 / (1.0 + jnp.exp(-sc_ref[...]))
    pt = jnp.where(p > THR, p, NEG)
    probs_ref[...] = pt
    pt3 = pt.reshape(B * C, NBLK, BLK)
    m = jnp.max(pt3, axis=-1)
    lane = lax.broadcasted_iota(jnp.int32, (B * C, NBLK, BLK), 2)
    il = jnp.min(jnp.where(pt3 == m[..., None], lane, _BIG_I), axis=-1)
    row = lax.broadcasted_iota(jnp.int32, (B * C, NBLK), 1)
    flat = il + row * BLK
    padf = jnp.full((B * C, NBLKP - NBLK), NEG, jnp.float32)
    padi = jnp.zeros((B * C, NBLKP + 16 - NBLK), jnp.int32)
    bmax_ref[...] = jnp.concatenate([m, padf], axis=-1)
    barg_ref[...] = jnp.concatenate([flat, padi], axis=-1)


def _tc_stage(rel_q, anch_q, scores_q):
    return pl.pallas_call(
        _tc_body,
        out_shape=[
            jax.ShapeDtypeStruct((4, B, N), jnp.float32),
            jax.ShapeDtypeStruct((B * C, N), jnp.float32),
            jax.ShapeDtypeStruct((B * C, NBLKP), jnp.float32),
            jax.ShapeDtypeStruct((B * C, NBLKP + 16), jnp.int32),
        ],
    )(rel_q, anch_q, scores_q)


_IOTA = functools.partial(lax.iota, jnp.int32, LANES)
_BIG_I = 1 << 30


def _vex_f(ref, idx):
    base = (idx // LANES) * LANES
    v = ref[pl.ds(base, LANES)]
    return lax.reduce_sum(jnp.where(_IOTA() == idx - base, v, 0.0), (0,))


def _vex_i(ref, idx):
    base = (idx // LANES) * LANES
    v = ref[pl.ds(base, LANES)]
    return lax.reduce_sum(jnp.where(_IOTA() == idx - base, v, 0), (0,))


def _vbro(ref, idx):
    base = (idx // LANES) * LANES
    v = ref[pl.ds(base, LANES)]
    lanes = jnp.full((LANES,), idx - base, jnp.int32)
    return v.at[lanes].get(mode="promise_in_bounds")


def _vset(ref, idx, val):
    base = (idx // LANES) * LANES
    v = ref[pl.ds(base, LANES)]
    ref[pl.ds(base, LANES)] = jnp.where(_IOTA() == idx - base, val, v)


def _scan_range(ref, start, nslices, unroll=False):
    def step(i, carry):
        vmax, vidx = carry
        off = start + i * LANES
        v = ref[pl.ds(off, LANES)]
        take = v > vmax
        return (jnp.where(take, v, vmax),
                jnp.where(take, off + _IOTA(), vidx))
    carry = (jnp.full((LANES,), NEG, jnp.float32), jnp.zeros((LANES,), jnp.int32))
    if unroll:
        for i in range(nslices):
            carry = step(i, carry)
        vmax, vidx = carry
    else:
        vmax, vidx = lax.fori_loop(0, nslices, step, carry)
    m = lax.reduce_max(vmax, (0,))
    idx = lax.reduce_min(jnp.where(vmax == m, vidx, _BIG_I), (0,))
    return m, idx


def _find_global(bm, nslices):
    return _scan_range(bm, 0, nslices, unroll=True)


def _nms_class(pv, py1, px1, py2, px2, bm, ba, sy1, sx1, sy2, sx2, ssc):
    zeros16 = jnp.zeros((LANES,), jnp.float32)
    negs16 = jnp.full((LANES,), NEG, jnp.float32)

    def init_sel(i, _):
        off = pl.ds(i * LANES, LANES)
        sy1[off] = zeros16
        sx1[off] = zeros16
        sy2[off] = zeros16
        sx2[off] = zeros16
        ssc[off] = negs16
        return 0
    lax.fori_loop(0, KP // LANES, init_sel, 0)

    g0, gb0 = _find_global(bm, NBLKP // LANES)

    def nms_cond(carry):
        nsel, gmax, _ = carry
        return (nsel < K) & (gmax > 0.0)

    def nms_body(carry):
        nsel, gmax, gblk = carry
        idx = _vex_i(ba, gblk)
        by1 = _vbro(py1, idx)
        bx1 = _vbro(px1, idx)
        by2 = _vbro(py2, idx)
        bx2 = _vbro(px2, idx)
        a1 = (by2 - by1) * (bx2 - bx1)

        def iou_step(i, acc):
            off = pl.ds(i * LANES, LANES)
            vy1 = sy1[off]
            vx1 = sx1[off]
            vy2 = sy2[off]
            vx2 = sx2[off]
            yy1 = jnp.maximum(by1, vy1)
            xx1 = jnp.maximum(bx1, vx1)
            yy2 = jnp.minimum(by2, vy2)
            xx2 = jnp.minimum(bx2, vx2)
            inter = (jnp.maximum(yy2 - yy1, 0.0)
                     * jnp.maximum(xx2 - xx1, 0.0))
            a2 = (vy2 - vy1) * (vx2 - vx1)
            iou = inter / (a1 + a2 - inter + 1e-8)
            return acc | (iou > IOU_THR)
        supm = jnp.zeros((LANES,), jnp.bool_)
        for i in range(KP // LANES):
            supm = iou_step(i, supm)
        sup = jnp.any(supm)

        _vset(sy1, nsel, jnp.where(sup, 0.0, by1))
        _vset(sx1, nsel, jnp.where(sup, 0.0, bx1))
        _vset(sy2, nsel, jnp.where(sup, 0.0, by2))
        _vset(sx2, nsel, jnp.where(sup, 0.0, bx2))
        _vset(ssc, nsel, jnp.where(sup, NEG, gmax))
        nsel = nsel + jnp.where(sup, 0, 1)

        _vset(pv, idx, NEG)
        m, fidx = _scan_range(pv, gblk * BLK, BLK // LANES, unroll=True)
        _vset(bm, gblk, m)
        _vset(ba, gblk, fidx)
        gmax2, gblk2 = _find_global(bm, NBLKP // LANES)
        return nsel, gmax2, gblk2

    lax.while_loop(nms_cond, nms_body, (jnp.int32(0), g0, gb0))


def _merge_batch(mc_y1, mc_x1, mc_y2, mc_x2, mc_sc, mbm, mba,
                 st_bx, st_sc, st_lb, st_nv):
    zeros16 = jnp.zeros((LANES,), jnp.float32)

    def init_mblk(b, _):
        m, idx = _scan_range(mc_sc, b * MBLK, MBLK // LANES)
        _vset(mbm, b, m)
        _vset(mba, b, idx)
        return 0
    lax.fori_loop(0, MTOT // MBLK, init_mblk, 0)

    def pad_mblk(i, _):
        _vset(mbm, MTOT // MBLK + i, NEG)
        _vset(mba, MTOT // MBLK + i, 0)
        return 0
    lax.fori_loop(0, 2 * LANES - MTOT // MBLK, pad_mblk, 0)

    def sel_step(i, nv):
        s, blk = _find_global(mbm, 2)
        f = _vex_i(mba, blk)
        valid = s > NEG / 2.0
        cc = f // KP
        by1 = _vbro(mc_y1, f)
        bx1 = _vbro(mc_x1, f)
        by2 = _vbro(mc_y2, f)
        bx2 = _vbro(mc_x2, f)
        sw = jnp.where(valid, s, 0.0)
        lw = jnp.where(valid, cc.astype(jnp.float32), 0.0)
        _vset(st_sc, i, sw)
        _vset(st_lb, i, lw)
        base = (i * 4 // LANES) * LANES
        off = i * 4 - base
        io = _IOTA()
        v = st_bx[pl.ds(base, LANES)]
        v = jnp.where(io == off, jnp.where(valid, by1, 0.0), v)
        v = jnp.where(io == off + 1, jnp.where(valid, bx1, 0.0), v)
        v = jnp.where(io == off + 2, jnp.where(valid, by2, 0.0), v)
        v = jnp.where(io == off + 3, jnp.where(valid, bx2, 0.0), v)
        st_bx[pl.ds(base, LANES)] = v
        _vset(mc_sc, f, NEG)
        m, fidx = _scan_range(mc_sc, blk * MBLK, MBLK // LANES, unroll=True)
        _vset(mbm, blk, m)
        _vset(mba, blk, fidx)
        return nv + jnp.where(valid, 1, 0)
    nv = lax.fori_loop(0, K, sel_step, jnp.int32(0))

    def pad_out(i, _):
        _vset(st_sc, K + i, 0.0)
        _vset(st_lb, K + i, 0.0)
        return 0
    lax.fori_loop(0, KP - K, pad_out, 0)

    def pad_bx(i, _):
        st_bx[pl.ds(K * 4 + i * LANES, LANES)] = zeros16
        return 0
    lax.fori_loop(0, (KP - K) * 4 // LANES, pad_bx, 0)

    st_nv[pl.ds(0, LANES)] = jnp.where(_IOTA() == 0, nv, 0)


def _sc_body(probs_hbm, boxes_hbm, bmax_hbm, barg_hbm, oboxes, oscores, olabels, onv,
             py1, px1, py2, px2, pv, bm, ba,
             sy1, sx1, sy2, sx2, ssc,
             sh_y1, sh_x1, sh_y2, sh_x2, sh_sc,
             mc_y1, mc_x1, mc_y2, mc_x2, mc_sc,
             mbm, mba, st_bx, st_sc, st_lb, st_nv):
    cidx = lax.axis_index("c")
    sidx = lax.axis_index("s")
    batch = cidx * 2 + sidx // 8
    j = sidx % 8
    bb = sidx // 8

    pltpu.sync_copy(boxes_hbm.at[0, batch], py1)
    pltpu.sync_copy(boxes_hbm.at[1, batch], px1)
    pltpu.sync_copy(boxes_hbm.at[2, batch], py2)
    pltpu.sync_copy(boxes_hbm.at[3, batch], px2)

    zeros16 = jnp.zeros((LANES,), jnp.float32)
    negs16 = jnp.full((LANES,), NEG, jnp.float32)

    for t in range(3):
        c = j + 8 * t

        @pl.when(c < C)
        def _():
            pltpu.sync_copy(probs_hbm.at[batch * C + c], pv)
            pltpu.sync_copy(bmax_hbm.at[batch * C + c], bm)
            pltpu.sync_copy(barg_hbm.at[batch * C + c], ba)
            _nms_class(pv, py1, px1, py2, px2, bm, ba, sy1, sx1, sy2, sx2, ssc)
            sh_off = bb * (C * KP + 16) + c * KP
            pltpu.sync_copy(sy1, sh_y1.at[pl.ds(sh_off, KP)])
            pltpu.sync_copy(sx1, sh_x1.at[pl.ds(sh_off, KP)])
            pltpu.sync_copy(sy2, sh_y2.at[pl.ds(sh_off, KP)])
            pltpu.sync_copy(sx2, sh_x2.at[pl.ds(sh_off, KP)])
            pltpu.sync_copy(ssc, sh_sc.at[pl.ds(sh_off, KP)])

    plsc.subcore_barrier()

    @pl.when(j == 5)
    def _():
        sh_b = bb * (C * KP + 16)
        pltpu.sync_copy(sh_y1.at[pl.ds(sh_b, C * KP + 16)], mc_y1)
        pltpu.sync_copy(sh_x1.at[pl.ds(sh_b, C * KP + 16)], mc_x1)
        pltpu.sync_copy(sh_y2.at[pl.ds(sh_b, C * KP + 16)], mc_y2)
        pltpu.sync_copy(sh_x2.at[pl.ds(sh_b, C * KP + 16)], mc_x2)
        pltpu.sync_copy(sh_sc.at[pl.ds(sh_b, C * KP)], mc_sc)

        _merge_batch(mc_y1, mc_x1, mc_y2, mc_x2, mc_sc, mbm, mba,
                     st_bx, st_sc, st_lb, st_nv)

        pltpu.sync_copy(st_bx, oboxes.at[batch])
        pltpu.sync_copy(st_sc, oscores.at[batch])
        pltpu.sync_copy(st_lb, olabels.at[batch])
        pltpu.sync_copy(st_nv, onv.at[batch])


def _sc_stage(probs, boxes_t, bmax, barg):
    mesh = plsc.VectorSubcoreMesh(core_axis_name="c", subcore_axis_name="s")
    f = pl.kernel(
        _sc_body,
        out_type=[
            jax.ShapeDtypeStruct((B, KP * 4), jnp.float32),
            jax.ShapeDtypeStruct((B, KP), jnp.float32),
            jax.ShapeDtypeStruct((B, KP), jnp.float32),
            jax.ShapeDtypeStruct((B, LANES), jnp.int32),
        ],
        mesh=mesh,
        compiler_params=pltpu.CompilerParams(needs_layout_passes=False),
        scratch_types=[
            pltpu.VMEM((N + 16,), jnp.float32),
            pltpu.VMEM((N + 16,), jnp.float32),
            pltpu.VMEM((N + 16,), jnp.float32),
            pltpu.VMEM((N + 16,), jnp.float32),
            pltpu.VMEM((N,), jnp.float32),
            pltpu.VMEM((NBLKP,), jnp.float32),
            pltpu.VMEM((NBLKP + 16,), jnp.int32),
            pltpu.VMEM((KP,), jnp.float32),
            pltpu.VMEM((KP,), jnp.float32),
            pltpu.VMEM((KP,), jnp.float32),
            pltpu.VMEM((KP,), jnp.float32),
            pltpu.VMEM((KP,), jnp.float32),
            pltpu.VMEM_SHARED((2 * (C * KP + 16),), jnp.float32),
            pltpu.VMEM_SHARED((2 * (C * KP + 16),), jnp.float32),
            pltpu.VMEM_SHARED((2 * (C * KP + 16),), jnp.float32),
            pltpu.VMEM_SHARED((2 * (C * KP + 16),), jnp.float32),
            pltpu.VMEM_SHARED((2 * (C * KP + 16),), jnp.float32),
            pltpu.VMEM((MTOT + 16,), jnp.float32),
            pltpu.VMEM((MTOT + 16,), jnp.float32),
            pltpu.VMEM((MTOT + 16,), jnp.float32),
            pltpu.VMEM((MTOT + 16,), jnp.float32),
            pltpu.VMEM((MTOT,), jnp.float32),
            pltpu.VMEM((2 * LANES,), jnp.float32),
            pltpu.VMEM((3 * LANES,), jnp.int32),
            pltpu.VMEM((KP * 4,), jnp.float32),
            pltpu.VMEM((KP,), jnp.float32),
            pltpu.VMEM((KP,), jnp.float32),
            pltpu.VMEM((LANES,), jnp.int32),
        ],
    )
    return f(probs, boxes_t, bmax, barg)


def kernel(rel_codes, scores, anchors):
    rel_q = jnp.transpose(rel_codes, (2, 0, 1))
    anch_q = jnp.transpose(anchors, (1, 0))
    scores_q = jnp.transpose(scores, (0, 2, 1)).reshape(B * C, N)
    boxes3d, probs, bmax, barg = _tc_stage(rel_q, anch_q, scores_q)
    boxes_t = jnp.pad(boxes3d, ((0, 0), (0, 0), (0, 16)))
    bx, sc, lb, nv = _sc_stage(probs, boxes_t, bmax, barg)
    out_boxes = bx.reshape(B, KP, 4)[:, :K, :]
    out_scores = sc[:, :K]
    out_labels = lb[:, :K]
    num_valid = nv[:, 0]
    return out_boxes, out_scores, out_labels, num_valid


# speedup vs baseline: 1.4758x; 1.0508x over previous
import functools
import jax
import jax.numpy as jnp
from jax import lax
from jax.experimental import pallas as pl
from jax.experimental.pallas import tpu as pltpu
from jax.experimental.pallas import tpu_sc as plsc

B = 4
N = 20000
C = 21
NEG = -1e9
THR = 0.3
IOU_THR = 0.5
K = 100
KP = 112
BLK = 160
NBLK = 125
NBLKP = 128
MBLK = 112
MTOT = 2352
LANES = 16
CP = 32


def _tc_body(rel_ref, anch_ref, sc_ref, boxes_ref, probs_ref, bmax_ref, barg_ref):
    ay1 = anch_ref[0]
    ax1 = anch_ref[1]
    ay2 = anch_ref[2]
    ax2 = anch_ref[3]
    ycenter_a = (ay1 + ay2) / 2.0
    xcenter_a = (ax1 + ax2) / 2.0
    ha = ay2 - ay1
    wa = ax2 - ax1
    ty = rel_ref[0] / 10.0
    tx = rel_ref[1] / 10.0
    th = rel_ref[2] / 5.0
    tw = rel_ref[3] / 5.0
    h = jnp.exp(th) * ha
    w = jnp.exp(tw) * wa
    yc = ty * ha + ycenter_a
    xc = tx * wa + xcenter_a
    boxes_ref[0] = jnp.clip(yc - h / 2.0, 0.0, 512.0)
    boxes_ref[1] = jnp.clip(xc - w / 2.0, 0.0, 512.0)
    boxes_ref[2] = jnp.clip(yc + h / 2.0, 0.0, 512.0)
    boxes_ref[3] = jnp.clip(xc + w / 2.0, 0.0, 512.0)
    p = 1.0 / (1.0 + jnp.exp(-sc_ref[...]))
    pt = jnp.where(p > THR, p, NEG)
    probs_ref[...] = pt
    pt3 = pt.reshape(B * CP, NBLK, BLK)
    m = jnp.max(pt3, axis=-1)
    lane = lax.broadcasted_iota(jnp.int32, (B * CP, NBLK, BLK), 2)
    il = jnp.min(jnp.where(pt3 == m[..., None], lane, _BIG_I), axis=-1)
    row = lax.broadcasted_iota(jnp.int32, (B * CP, NBLK), 1)
    flat = il + row * BLK
    padf = jnp.full((B * CP, NBLKP - NBLK), NEG, jnp.float32)
    padi = jnp.zeros((B * CP, NBLKP + 16 - NBLK), jnp.int32)
    bmax_ref[...] = jnp.concatenate([m, padf], axis=-1)
    barg_ref[...] = jnp.concatenate([flat, padi], axis=-1)


def _tc_stage(rel_q, anch_q, scores_q):
    return pl.pallas_call(
        _tc_body,
        out_shape=[
            jax.ShapeDtypeStruct((4, B, N), jnp.float32),
            jax.ShapeDtypeStruct((B * CP, N), jnp.float32),
            jax.ShapeDtypeStruct((B * CP, NBLKP), jnp.float32),
            jax.ShapeDtypeStruct((B * CP, NBLKP + 16), jnp.int32),
        ],
    )(rel_q, anch_q, scores_q)


_IOTA = functools.partial(lax.iota, jnp.int32, LANES)
_BIG_I = 1 << 30


def _vex_f(ref, idx):
    base = (idx // LANES) * LANES
    v = ref[pl.ds(base, LANES)]
    return lax.reduce_sum(jnp.where(_IOTA() == idx - base, v, 0.0), (0,))


def _vex_i(ref, idx):
    base = (idx // LANES) * LANES
    v = ref[pl.ds(base, LANES)]
    return lax.reduce_sum(jnp.where(_IOTA() == idx - base, v, 0), (0,))


def _vbro(ref, idx):
    base = (idx // LANES) * LANES
    v = ref[pl.ds(base, LANES)]
    lanes = jnp.full((LANES,), idx - base, jnp.int32)
    return v.at[lanes].get(mode="promise_in_bounds")


def _vset(ref, idx, val):
    base = (idx // LANES) * LANES
    v = ref[pl.ds(base, LANES)]
    ref[pl.ds(base, LANES)] = jnp.where(_IOTA() == idx - base, val, v)


def _scan_range(ref, start, nslices, unroll=False):
    def step(i, carry):
        vmax, vidx = carry
        off = start + i * LANES
        v = ref[pl.ds(off, LANES)]
        take = v > vmax
        return (jnp.where(take, v, vmax),
                jnp.where(take, off + _IOTA(), vidx))
    carry = (jnp.full((LANES,), NEG, jnp.float32), jnp.zeros((LANES,), jnp.int32))
    if unroll:
        for i in range(nslices):
            carry = step(i, carry)
        vmax, vidx = carry
    else:
        vmax, vidx = lax.fori_loop(0, nslices, step, carry)
    m = lax.reduce_max(vmax, (0,))
    idx = lax.reduce_min(jnp.where(vmax == m, vidx, _BIG_I), (0,))
    return m, idx


def _find_global(bm, nslices):
    return _scan_range(bm, 0, nslices, unroll=True)


def _nms_class(pv, py1, px1, py2, px2, bm, ba, sy1, sx1, sy2, sx2, ssc):
    zeros16 = jnp.zeros((LANES,), jnp.float32)
    negs16 = jnp.full((LANES,), NEG, jnp.float32)

    def init_sel(i, _):
        off = pl.ds(i * LANES, LANES)
        sy1[off] = zeros16
        sx1[off] = zeros16
        sy2[off] = zeros16
        sx2[off] = zeros16
        ssc[off] = negs16
        return 0
    lax.fori_loop(0, KP // LANES, init_sel, 0)

    g0, gb0 = _find_global(bm, NBLKP // LANES)

    def nms_cond(carry):
        nsel, gmax, _ = carry
        return (nsel < K) & (gmax > 0.0)

    def nms_body(carry):
        nsel, gmax, gblk = carry
        idx = _vex_i(ba, gblk)
        by1 = _vbro(py1, idx)
        bx1 = _vbro(px1, idx)
        by2 = _vbro(py2, idx)
        bx2 = _vbro(px2, idx)
        a1 = (by2 - by1) * (bx2 - bx1)

        def iou_step(i, acc):
            off = pl.ds(i * LANES, LANES)
            vy1 = sy1[off]
            vx1 = sx1[off]
            vy2 = sy2[off]
            vx2 = sx2[off]
            yy1 = jnp.maximum(by1, vy1)
            xx1 = jnp.maximum(bx1, vx1)
            yy2 = jnp.minimum(by2, vy2)
            xx2 = jnp.minimum(bx2, vx2)
            inter = (jnp.maximum(yy2 - yy1, 0.0)
                     * jnp.maximum(xx2 - xx1, 0.0))
            a2 = (vy2 - vy1) * (vx2 - vx1)
            iou = inter / (a1 + a2 - inter + 1e-8)
            return acc | (iou > IOU_THR)
        supm = jnp.zeros((LANES,), jnp.bool_)
        for i in range(KP // LANES):
            supm = iou_step(i, supm)
        sup = jnp.any(supm)

        _vset(sy1, nsel, jnp.where(sup, 0.0, by1))
        _vset(sx1, nsel, jnp.where(sup, 0.0, bx1))
        _vset(sy2, nsel, jnp.where(sup, 0.0, by2))
        _vset(sx2, nsel, jnp.where(sup, 0.0, bx2))
        _vset(ssc, nsel, jnp.where(sup, NEG, gmax))
        nsel = nsel + jnp.where(sup, 0, 1)

        _vset(pv, idx, NEG)
        m, fidx = _scan_range(pv, gblk * BLK, BLK // LANES, unroll=True)
        _vset(bm, gblk, m)
        _vset(ba, gblk, fidx)
        gmax2, gblk2 = _find_global(bm, NBLKP // LANES)
        return nsel, gmax2, gblk2

    lax.while_loop(nms_cond, nms_body, (jnp.int32(0), g0, gb0))


def _merge_batch(mc_y1, mc_x1, mc_y2, mc_x2, mc_sc, mbm, mba,
                 st_bx, st_sc, st_lb, st_nv):
    zeros16 = jnp.zeros((LANES,), jnp.float32)

    def init_mblk(b, _):
        m, idx = _scan_range(mc_sc, b * MBLK, MBLK // LANES)
        _vset(mbm, b, m)
        _vset(mba, b, idx)
        return 0
    lax.fori_loop(0, MTOT // MBLK, init_mblk, 0)

    def pad_mblk(i, _):
        _vset(mbm, MTOT // MBLK + i, NEG)
        _vset(mba, MTOT // MBLK + i, 0)
        return 0
    lax.fori_loop(0, 2 * LANES - MTOT // MBLK, pad_mblk, 0)

    def sel_step(i, nv):
        s, blk = _find_global(mbm, 2)
        f = _vex_i(mba, blk)
        valid = s > NEG / 2.0
        cc = f // KP
        by1 = _vbro(mc_y1, f)
        bx1 = _vbro(mc_x1, f)
        by2 = _vbro(mc_y2, f)
        bx2 = _vbro(mc_x2, f)
        sw = jnp.where(valid, s, 0.0)
        lw = jnp.where(valid, cc.astype(jnp.float32), 0.0)
        _vset(st_sc, i, sw)
        _vset(st_lb, i, lw)
        base = (i * 4 // LANES) * LANES
        off = i * 4 - base
        io = _IOTA()
        v = st_bx[pl.ds(base, LANES)]
        v = jnp.where(io == off, jnp.where(valid, by1, 0.0), v)
        v = jnp.where(io == off + 1, jnp.where(valid, bx1, 0.0), v)
        v = jnp.where(io == off + 2, jnp.where(valid, by2, 0.0), v)
        v = jnp.where(io == off + 3, jnp.where(valid, bx2, 0.0), v)
        st_bx[pl.ds(base, LANES)] = v
        _vset(mc_sc, f, NEG)
        m, fidx = _scan_range(mc_sc, blk * MBLK, MBLK // LANES, unroll=True)
        _vset(mbm, blk, m)
        _vset(mba, blk, fidx)
        return nv + jnp.where(valid, 1, 0)
    nv = lax.fori_loop(0, K, sel_step, jnp.int32(0))

    def pad_out(i, _):
        _vset(st_sc, K + i, 0.0)
        _vset(st_lb, K + i, 0.0)
        return 0
    lax.fori_loop(0, KP - K, pad_out, 0)

    def pad_bx(i, _):
        st_bx[pl.ds(K * 4 + i * LANES, LANES)] = zeros16
        return 0
    lax.fori_loop(0, (KP - K) * 4 // LANES, pad_bx, 0)

    st_nv[pl.ds(0, LANES)] = jnp.where(_IOTA() == 0, nv, 0)


def _sc_body(probs_hbm, boxes_hbm, bmax_hbm, barg_hbm, oboxes, oscores, olabels, onv,
             py1, px1, py2, px2, pv, bm, ba,
             sy1, sx1, sy2, sx2, ssc,
             sh_y1, sh_x1, sh_y2, sh_x2, sh_sc,
             mc_y1, mc_x1, mc_y2, mc_x2, mc_sc,
             mbm, mba, st_bx, st_sc, st_lb, st_nv):
    cidx = lax.axis_index("c")
    sidx = lax.axis_index("s")
    batch = cidx * 2 + sidx // 8
    j = sidx % 8
    bb = sidx // 8

    pltpu.sync_copy(boxes_hbm.at[0, batch], py1)
    pltpu.sync_copy(boxes_hbm.at[1, batch], px1)
    pltpu.sync_copy(boxes_hbm.at[2, batch], py2)
    pltpu.sync_copy(boxes_hbm.at[3, batch], px2)

    zeros16 = jnp.zeros((LANES,), jnp.float32)
    negs16 = jnp.full((LANES,), NEG, jnp.float32)

    for t in range(3):
        c = j + 8 * t

        @pl.when(c < C)
        def _():
            pltpu.sync_copy(probs_hbm.at[batch * CP + c], pv)
            pltpu.sync_copy(bmax_hbm.at[batch * CP + c], bm)
            pltpu.sync_copy(barg_hbm.at[batch * CP + c], ba)
            _nms_class(pv, py1, px1, py2, px2, bm, ba, sy1, sx1, sy2, sx2, ssc)
            sh_off = bb * (C * KP + 16) + c * KP
            pltpu.sync_copy(sy1, sh_y1.at[pl.ds(sh_off, KP)])
            pltpu.sync_copy(sx1, sh_x1.at[pl.ds(sh_off, KP)])
            pltpu.sync_copy(sy2, sh_y2.at[pl.ds(sh_off, KP)])
            pltpu.sync_copy(sx2, sh_x2.at[pl.ds(sh_off, KP)])
            pltpu.sync_copy(ssc, sh_sc.at[pl.ds(sh_off, KP)])

    plsc.subcore_barrier()

    @pl.when(j == 5)
    def _():
        sh_b = bb * (C * KP + 16)
        pltpu.sync_copy(sh_y1.at[pl.ds(sh_b, C * KP + 16)], mc_y1)
        pltpu.sync_copy(sh_x1.at[pl.ds(sh_b, C * KP + 16)], mc_x1)
        pltpu.sync_copy(sh_y2.at[pl.ds(sh_b, C * KP + 16)], mc_y2)
        pltpu.sync_copy(sh_x2.at[pl.ds(sh_b, C * KP + 16)], mc_x2)
        pltpu.sync_copy(sh_sc.at[pl.ds(sh_b, C * KP)], mc_sc)

        _merge_batch(mc_y1, mc_x1, mc_y2, mc_x2, mc_sc, mbm, mba,
                     st_bx, st_sc, st_lb, st_nv)

        pltpu.sync_copy(st_bx, oboxes.at[batch])
        pltpu.sync_copy(st_sc, oscores.at[batch])
        pltpu.sync_copy(st_lb, olabels.at[batch])
        pltpu.sync_copy(st_nv, onv.at[batch])


def _sc_stage(probs, boxes_t, bmax, barg):
    mesh = plsc.VectorSubcoreMesh(core_axis_name="c", subcore_axis_name="s")
    f = pl.kernel(
        _sc_body,
        out_type=[
            jax.ShapeDtypeStruct((B, KP * 4), jnp.float32),
            jax.ShapeDtypeStruct((B, KP), jnp.float32),
            jax.ShapeDtypeStruct((B, KP), jnp.float32),
            jax.ShapeDtypeStruct((B, LANES), jnp.int32),
        ],
        mesh=mesh,
        compiler_params=pltpu.CompilerParams(needs_layout_passes=False),
        scratch_types=[
            pltpu.VMEM((N + 16,), jnp.float32),
            pltpu.VMEM((N + 16,), jnp.float32),
            pltpu.VMEM((N + 16,), jnp.float32),
            pltpu.VMEM((N + 16,), jnp.float32),
            pltpu.VMEM((N,), jnp.float32),
            pltpu.VMEM((NBLKP,), jnp.float32),
            pltpu.VMEM((NBLKP + 16,), jnp.int32),
            pltpu.VMEM((KP,), jnp.float32),
            pltpu.VMEM((KP,), jnp.float32),
            pltpu.VMEM((KP,), jnp.float32),
            pltpu.VMEM((KP,), jnp.float32),
            pltpu.VMEM((KP,), jnp.float32),
            pltpu.VMEM_SHARED((2 * (C * KP + 16),), jnp.float32),
            pltpu.VMEM_SHARED((2 * (C * KP + 16),), jnp.float32),
            pltpu.VMEM_SHARED((2 * (C * KP + 16),), jnp.float32),
            pltpu.VMEM_SHARED((2 * (C * KP + 16),), jnp.float32),
            pltpu.VMEM_SHARED((2 * (C * KP + 16),), jnp.float32),
            pltpu.VMEM((MTOT + 16,), jnp.float32),
            pltpu.VMEM((MTOT + 16,), jnp.float32),
            pltpu.VMEM((MTOT + 16,), jnp.float32),
            pltpu.VMEM((MTOT + 16,), jnp.float32),
            pltpu.VMEM((MTOT,), jnp.float32),
            pltpu.VMEM((2 * LANES,), jnp.float32),
            pltpu.VMEM((3 * LANES,), jnp.int32),
            pltpu.VMEM((KP * 4,), jnp.float32),
            pltpu.VMEM((KP,), jnp.float32),
            pltpu.VMEM((KP,), jnp.float32),
            pltpu.VMEM((LANES,), jnp.int32),
        ],
    )
    return f(probs, boxes_t, bmax, barg)


def kernel(rel_codes, scores, anchors):
    rel_q = jnp.transpose(rel_codes, (2, 0, 1))
    anch_q = jnp.transpose(anchors, (1, 0))
    scores_p = jnp.pad(scores, ((0, 0), (0, 0), (0, CP - C)))
    scores_q = jnp.transpose(scores_p, (0, 2, 1)).reshape(B * CP, N)
    boxes3d, probs, bmax, barg = _tc_stage(rel_q, anch_q, scores_q)
    boxes_t = jnp.pad(boxes3d, ((0, 0), (0, 0), (0, 16)))
    bx, sc, lb, nv = _sc_stage(probs, boxes_t, bmax, barg)
    out_boxes = bx.reshape(B, KP, 4)[:, :K, :]
    out_scores = sc[:, :K]
    out_labels = lb[:, :K]
    num_valid = nv[:, 0]
    return out_boxes, out_scores, out_labels, num_valid


# speedup vs baseline: 1.5060x; 1.0205x over previous
import functools
import jax
import jax.numpy as jnp
from jax import lax
from jax.experimental import pallas as pl
from jax.experimental.pallas import tpu as pltpu
from jax.experimental.pallas import tpu_sc as plsc

B = 4
N = 20000
C = 21
NEG = -1e9
THR = 0.3
IOU_THR = 0.5
K = 100
KP = 112
BLK = 400
NBLK = 50
NBLKP = 64
RCH = 16
MBLK = 112
MTOT = 2352
LANES = 16
CP = 32


def _tc_decode_body(rel_ref, anch_ref, boxes_ref):
    ay1 = anch_ref[0]
    ax1 = anch_ref[1]
    ay2 = anch_ref[2]
    ax2 = anch_ref[3]
    ycenter_a = (ay1 + ay2) / 2.0
    xcenter_a = (ax1 + ax2) / 2.0
    ha = ay2 - ay1
    wa = ax2 - ax1
    ty = rel_ref[0] / 10.0
    tx = rel_ref[1] / 10.0
    th = rel_ref[2] / 5.0
    tw = rel_ref[3] / 5.0
    h = jnp.exp(th) * ha
    w = jnp.exp(tw) * wa
    yc = ty * ha + ycenter_a
    xc = tx * wa + xcenter_a
    boxes_ref[0] = jnp.clip(yc - h / 2.0, 0.0, 512.0)
    boxes_ref[1] = jnp.clip(xc - w / 2.0, 0.0, 512.0)
    boxes_ref[2] = jnp.clip(yc + h / 2.0, 0.0, 512.0)
    boxes_ref[3] = jnp.clip(xc + w / 2.0, 0.0, 512.0)


def _tc_scores_body(sc_ref, probs_ref, bmax_ref, barg_ref):
    p = 1.0 / (1.0 + jnp.exp(-sc_ref[...]))
    pt = jnp.where(p > THR, p, NEG)
    probs_ref[...] = pt
    pt3 = pt.reshape(RCH, NBLK, BLK)
    m = jnp.max(pt3, axis=-1)
    lane = lax.broadcasted_iota(jnp.int32, (RCH, NBLK, BLK), 2)
    il = jnp.min(jnp.where(pt3 == m[..., None], lane, _BIG_I), axis=-1)
    row = lax.broadcasted_iota(jnp.int32, (RCH, NBLK), 1)
    padf = jnp.full((RCH, 128 - NBLK), NEG, jnp.float32)
    padi = jnp.zeros((RCH, 128 - NBLK), jnp.int32)
    bmax_ref[...] = jnp.concatenate([m, padf], axis=-1)
    barg_ref[...] = jnp.concatenate([il + row * BLK, padi], axis=-1)


def _tc_stage(rel_q, anch_q, scores_q):
    boxes = pl.pallas_call(
        _tc_decode_body,
        out_shape=jax.ShapeDtypeStruct((4, B, N), jnp.float32),
    )(rel_q, anch_q)
    probs, bmax, barg = pl.pallas_call(
        _tc_scores_body,
        grid=(B * CP // RCH,),
        in_specs=[pl.BlockSpec((RCH, N), lambda i: (i, 0))],
        out_specs=[
            pl.BlockSpec((RCH, N), lambda i: (i, 0)),
            pl.BlockSpec((RCH, 128), lambda i: (i, 0)),
            pl.BlockSpec((RCH, 128), lambda i: (i, 0)),
        ],
        out_shape=[
            jax.ShapeDtypeStruct((B * CP, N), jnp.float32),
            jax.ShapeDtypeStruct((B * CP, 128), jnp.float32),
            jax.ShapeDtypeStruct((B * CP, 128), jnp.int32),
        ],
    )(scores_q)
    return boxes, probs, bmax, barg


_IOTA = functools.partial(lax.iota, jnp.int32, LANES)
_BIG_I = 1 << 30


def _vex_f(ref, idx):
    base = (idx // LANES) * LANES
    v = ref[pl.ds(base, LANES)]
    return lax.reduce_sum(jnp.where(_IOTA() == idx - base, v, 0.0), (0,))


def _vex_i(ref, idx):
    base = (idx // LANES) * LANES
    v = ref[pl.ds(base, LANES)]
    return lax.reduce_sum(jnp.where(_IOTA() == idx - base, v, 0), (0,))


def _vbro(ref, idx):
    base = (idx // LANES) * LANES
    v = ref[pl.ds(base, LANES)]
    lanes = jnp.full((LANES,), idx - base, jnp.int32)
    return v.at[lanes].get(mode="promise_in_bounds")


def _vset(ref, idx, val):
    base = (idx // LANES) * LANES
    v = ref[pl.ds(base, LANES)]
    ref[pl.ds(base, LANES)] = jnp.where(_IOTA() == idx - base, val, v)


def _scan_range(ref, start, nslices, unroll=False):
    def step(i, carry):
        vmax, vidx = carry
        off = start + i * LANES
        v = ref[pl.ds(off, LANES)]
        take = v > vmax
        return (jnp.where(take, v, vmax),
                jnp.where(take, off + _IOTA(), vidx))
    carry = (jnp.full((LANES,), NEG, jnp.float32), jnp.zeros((LANES,), jnp.int32))
    if unroll:
        for i in range(nslices):
            carry = step(i, carry)
        vmax, vidx = carry
    else:
        vmax, vidx = lax.fori_loop(0, nslices, step, carry)
    m = lax.reduce_max(vmax, (0,))
    idx = lax.reduce_min(jnp.where(vmax == m, vidx, _BIG_I), (0,))
    return m, idx


def _find_global(bm, nslices):
    return _scan_range(bm, 0, nslices, unroll=True)


def _nms_class(pv, py1, px1, py2, px2, bm, ba, sy1, sx1, sy2, sx2, ssc):
    zeros16 = jnp.zeros((LANES,), jnp.float32)
    negs16 = jnp.full((LANES,), NEG, jnp.float32)

    def init_sel(i, _):
        off = pl.ds(i * LANES, LANES)
        sy1[off] = zeros16
        sx1[off] = zeros16
        sy2[off] = zeros16
        sx2[off] = zeros16
        ssc[off] = negs16
        return 0
    lax.fori_loop(0, KP // LANES, init_sel, 0)

    g0, gb0 = _find_global(bm, NBLKP // LANES)

    def nms_cond(carry):
        nsel, gmax, _ = carry
        return (nsel < K) & (gmax > 0.0)

    def nms_body(carry):
        nsel, gmax, gblk = carry
        idx = _vex_i(ba, gblk)
        by1 = _vbro(py1, idx)
        bx1 = _vbro(px1, idx)
        by2 = _vbro(py2, idx)
        bx2 = _vbro(px2, idx)
        a1 = (by2 - by1) * (bx2 - bx1)

        def iou_step(i, acc):
            off = pl.ds(i * LANES, LANES)
            vy1 = sy1[off]
            vx1 = sx1[off]
            vy2 = sy2[off]
            vx2 = sx2[off]
            yy1 = jnp.maximum(by1, vy1)
            xx1 = jnp.maximum(bx1, vx1)
            yy2 = jnp.minimum(by2, vy2)
            xx2 = jnp.minimum(bx2, vx2)
            inter = (jnp.maximum(yy2 - yy1, 0.0)
                     * jnp.maximum(xx2 - xx1, 0.0))
            a2 = (vy2 - vy1) * (vx2 - vx1)
            iou = inter / (a1 + a2 - inter + 1e-8)
            return acc | (iou > IOU_THR)
        supm = jnp.zeros((LANES,), jnp.bool_)
        for i in range(KP // LANES):
            supm = iou_step(i, supm)
        sup = jnp.any(supm)

        _vset(sy1, nsel, jnp.where(sup, 0.0, by1))
        _vset(sx1, nsel, jnp.where(sup, 0.0, bx1))
        _vset(sy2, nsel, jnp.where(sup, 0.0, by2))
        _vset(sx2, nsel, jnp.where(sup, 0.0, bx2))
        _vset(ssc, nsel, jnp.where(sup, NEG, gmax))
        nsel = nsel + jnp.where(sup, 0, 1)

        _vset(pv, idx, NEG)
        m, fidx = _scan_range(pv, gblk * BLK, BLK // LANES, unroll=True)
        _vset(bm, gblk, m)
        _vset(ba, gblk, fidx)
        gmax2, gblk2 = _find_global(bm, NBLKP // LANES)
        return nsel, gmax2, gblk2

    lax.while_loop(nms_cond, nms_body, (jnp.int32(0), g0, gb0))


def _merge_batch(mc_y1, mc_x1, mc_y2, mc_x2, mc_sc, mbm, mba,
                 st_bx, st_sc, st_lb, st_nv):
    zeros16 = jnp.zeros((LANES,), jnp.float32)

    def init_mblk(b, _):
        m, idx = _scan_range(mc_sc, b * MBLK, MBLK // LANES)
        _vset(mbm, b, m)
        _vset(mba, b, idx)
        return 0
    lax.fori_loop(0, MTOT // MBLK, init_mblk, 0)

    def pad_mblk(i, _):
        _vset(mbm, MTOT // MBLK + i, NEG)
        _vset(mba, MTOT // MBLK + i, 0)
        return 0
    lax.fori_loop(0, 2 * LANES - MTOT // MBLK, pad_mblk, 0)

    def sel_step(i, nv):
        s, blk = _find_global(mbm, 2)
        f = _vex_i(mba, blk)
        valid = s > NEG / 2.0
        cc = f // KP
        by1 = _vbro(mc_y1, f)
        bx1 = _vbro(mc_x1, f)
        by2 = _vbro(mc_y2, f)
        bx2 = _vbro(mc_x2, f)
        sw = jnp.where(valid, s, 0.0)
        lw = jnp.where(valid, cc.astype(jnp.float32), 0.0)
        _vset(st_sc, i, sw)
        _vset(st_lb, i, lw)
        base = (i * 4 // LANES) * LANES
        off = i * 4 - base
        io = _IOTA()
        v = st_bx[pl.ds(base, LANES)]
        v = jnp.where(io == off, jnp.where(valid, by1, 0.0), v)
        v = jnp.where(io == off + 1, jnp.where(valid, bx1, 0.0), v)
        v = jnp.where(io == off + 2, jnp.where(valid, by2, 0.0), v)
        v = jnp.where(io == off + 3, jnp.where(valid, bx2, 0.0), v)
        st_bx[pl.ds(base, LANES)] = v
        _vset(mc_sc, f, NEG)
        m, fidx = _scan_range(mc_sc, blk * MBLK, MBLK // LANES, unroll=True)
        _vset(mbm, blk, m)
        _vset(mba, blk, fidx)
        return nv + jnp.where(valid, 1, 0)
    nv = lax.fori_loop(0, K, sel_step, jnp.int32(0))

    def pad_out(i, _):
        _vset(st_sc, K + i, 0.0)
        _vset(st_lb, K + i, 0.0)
        return 0
    lax.fori_loop(0, KP - K, pad_out, 0)

    def pad_bx(i, _):
        st_bx[pl.ds(K * 4 + i * LANES, LANES)] = zeros16
        return 0
    lax.fori_loop(0, (KP - K) * 4 // LANES, pad_bx, 0)

    st_nv[pl.ds(0, LANES)] = jnp.where(_IOTA() == 0, nv, 0)


def _sc_body(probs_hbm, boxes_hbm, bmax_hbm, barg_hbm, oboxes, oscores, olabels, onv,
             py1, px1, py2, px2, pv, bm, ba,
             sy1, sx1, sy2, sx2, ssc,
             sh_y1, sh_x1, sh_y2, sh_x2, sh_sc,
             mc_y1, mc_x1, mc_y2, mc_x2, mc_sc,
             mbm, mba, st_bx, st_sc, st_lb, st_nv):
    cidx = lax.axis_index("c")
    sidx = lax.axis_index("s")
    batch = cidx * 2 + sidx // 8
    j = sidx % 8
    bb = sidx // 8

    pltpu.sync_copy(boxes_hbm.at[0, batch], py1)
    pltpu.sync_copy(boxes_hbm.at[1, batch], px1)
    pltpu.sync_copy(boxes_hbm.at[2, batch], py2)
    pltpu.sync_copy(boxes_hbm.at[3, batch], px2)

    zeros16 = jnp.zeros((LANES,), jnp.float32)
    negs16 = jnp.full((LANES,), NEG, jnp.float32)

    for t in range(3):
        c = j + 8 * t

        @pl.when(c < C)
        def _():
            pltpu.sync_copy(probs_hbm.at[batch * CP + c], pv)
            pltpu.sync_copy(bmax_hbm.at[batch * CP + c], bm)
            pltpu.sync_copy(barg_hbm.at[batch * CP + c], ba.at[pl.ds(0, 128)])
            _nms_class(pv, py1, px1, py2, px2, bm, ba, sy1, sx1, sy2, sx2, ssc)
            sh_off = bb * (C * KP + 16) + c * KP
            pltpu.sync_copy(sy1, sh_y1.at[pl.ds(sh_off, KP)])
            pltpu.sync_copy(sx1, sh_x1.at[pl.ds(sh_off, KP)])
            pltpu.sync_copy(sy2, sh_y2.at[pl.ds(sh_off, KP)])
            pltpu.sync_copy(sx2, sh_x2.at[pl.ds(sh_off, KP)])
            pltpu.sync_copy(ssc, sh_sc.at[pl.ds(sh_off, KP)])

    plsc.subcore_barrier()

    @pl.when(j == 5)
    def _():
        sh_b = bb * (C * KP + 16)
        pltpu.sync_copy(sh_y1.at[pl.ds(sh_b, C * KP + 16)], mc_y1)
        pltpu.sync_copy(sh_x1.at[pl.ds(sh_b, C * KP + 16)], mc_x1)
        pltpu.sync_copy(sh_y2.at[pl.ds(sh_b, C * KP + 16)], mc_y2)
        pltpu.sync_copy(sh_x2.at[pl.ds(sh_b, C * KP + 16)], mc_x2)
        pltpu.sync_copy(sh_sc.at[pl.ds(sh_b, C * KP)], mc_sc)

        _merge_batch(mc_y1, mc_x1, mc_y2, mc_x2, mc_sc, mbm, mba,
                     st_bx, st_sc, st_lb, st_nv)

        pltpu.sync_copy(st_bx, oboxes.at[batch])
        pltpu.sync_copy(st_sc, oscores.at[batch])
        pltpu.sync_copy(st_lb, olabels.at[batch])
        pltpu.sync_copy(st_nv, onv.at[batch])


def _sc_stage(probs, boxes_t, bmax, barg):
    mesh = plsc.VectorSubcoreMesh(core_axis_name="c", subcore_axis_name="s")
    f = pl.kernel(
        _sc_body,
        out_type=[
            jax.ShapeDtypeStruct((B, KP * 4), jnp.float32),
            jax.ShapeDtypeStruct((B, KP), jnp.float32),
            jax.ShapeDtypeStruct((B, KP), jnp.float32),
            jax.ShapeDtypeStruct((B, LANES), jnp.int32),
        ],
        mesh=mesh,
        compiler_params=pltpu.CompilerParams(needs_layout_passes=False),
        scratch_types=[
            pltpu.VMEM((N + 16,), jnp.float32),
            pltpu.VMEM((N + 16,), jnp.float32),
            pltpu.VMEM((N + 16,), jnp.float32),
            pltpu.VMEM((N + 16,), jnp.float32),
            pltpu.VMEM((N,), jnp.float32),
            pltpu.VMEM((128,), jnp.float32),
            pltpu.VMEM((128,), jnp.int32),
            pltpu.VMEM((KP,), jnp.float32),
            pltpu.VMEM((KP,), jnp.float32),
            pltpu.VMEM((KP,), jnp.float32),
            pltpu.VMEM((KP,), jnp.float32),
            pltpu.VMEM((KP,), jnp.float32),
            pltpu.VMEM_SHARED((2 * (C * KP + 16),), jnp.float32),
            pltpu.VMEM_SHARED((2 * (C * KP + 16),), jnp.float32),
            pltpu.VMEM_SHARED((2 * (C * KP + 16),), jnp.float32),
            pltpu.VMEM_SHARED((2 * (C * KP + 16),), jnp.float32),
            pltpu.VMEM_SHARED((2 * (C * KP + 16),), jnp.float32),
            pltpu.VMEM((MTOT + 16,), jnp.float32),
            pltpu.VMEM((MTOT + 16,), jnp.float32),
            pltpu.VMEM((MTOT + 16,), jnp.float32),
            pltpu.VMEM((MTOT + 16,), jnp.float32),
            pltpu.VMEM((MTOT,), jnp.float32),
            pltpu.VMEM((2 * LANES,), jnp.float32),
            pltpu.VMEM((3 * LANES,), jnp.int32),
            pltpu.VMEM((KP * 4,), jnp.float32),
            pltpu.VMEM((KP,), jnp.float32),
            pltpu.VMEM((KP,), jnp.float32),
            pltpu.VMEM((LANES,), jnp.int32),
        ],
    )
    return f(probs, boxes_t, bmax, barg)


def kernel(rel_codes, scores, anchors):
    rel_q = jnp.transpose(rel_codes, (2, 0, 1))
    anch_q = jnp.transpose(anchors, (1, 0))
    scores_p = jnp.pad(scores, ((0, 0), (0, 0), (0, CP - C)))
    scores_q = jnp.transpose(scores_p, (0, 2, 1)).reshape(B * CP, N)
    boxes3d, probs, bmax, barg = _tc_stage(rel_q, anch_q, scores_q)
    boxes_t = jnp.pad(boxes3d, ((0, 0), (0, 0), (0, 16)))
    bx, sc, lb, nv = _sc_stage(probs, boxes_t, bmax, barg)
    out_boxes = bx.reshape(B, KP, 4)[:, :K, :]
    out_scores = sc[:, :K]
    out_labels = lb[:, :K]
    num_valid = nv[:, 0]
    return out_boxes, out_scores, out_labels, num_valid


# speedup vs baseline: 1.5334x; 1.0182x over previous
import functools
import jax
import jax.numpy as jnp
from jax import lax
from jax.experimental import pallas as pl
from jax.experimental.pallas import tpu as pltpu
from jax.experimental.pallas import tpu_sc as plsc

B = 4
N = 20000
C = 21
NEG = -1e9
THR = 0.3
IOU_THR = 0.5
K = 100
KP = 112
BLK = 160
NBLK = 125
NBLKP = 128
RCH = 16
MBLK = 112
MTOT = 2352
LANES = 16
CP = 32


def _tc_decode_body(rel_ref, anch_ref, boxes_ref):
    ay1 = anch_ref[0]
    ax1 = anch_ref[1]
    ay2 = anch_ref[2]
    ax2 = anch_ref[3]
    ycenter_a = (ay1 + ay2) / 2.0
    xcenter_a = (ax1 + ax2) / 2.0
    ha = ay2 - ay1
    wa = ax2 - ax1
    ty = rel_ref[0] / 10.0
    tx = rel_ref[1] / 10.0
    th = rel_ref[2] / 5.0
    tw = rel_ref[3] / 5.0
    h = jnp.exp(th) * ha
    w = jnp.exp(tw) * wa
    yc = ty * ha + ycenter_a
    xc = tx * wa + xcenter_a
    boxes_ref[0] = jnp.clip(yc - h / 2.0, 0.0, 512.0)
    boxes_ref[1] = jnp.clip(xc - w / 2.0, 0.0, 512.0)
    boxes_ref[2] = jnp.clip(yc + h / 2.0, 0.0, 512.0)
    boxes_ref[3] = jnp.clip(xc + w / 2.0, 0.0, 512.0)


def _tc_scores_body(sc_ref, probs_ref, bmax_ref, barg_ref):
    p = 1.0 / (1.0 + jnp.exp(-sc_ref[...]))
    pt = jnp.where(p > THR, p, NEG)
    probs_ref[...] = pt
    pt3 = pt.reshape(RCH, NBLK, BLK)
    m = jnp.max(pt3, axis=-1)
    lane = lax.broadcasted_iota(jnp.int32, (RCH, NBLK, BLK), 2)
    il = jnp.min(jnp.where(pt3 == m[..., None], lane, _BIG_I), axis=-1)
    row = lax.broadcasted_iota(jnp.int32, (RCH, NBLK), 1)
    padf = jnp.full((RCH, 128 - NBLK), NEG, jnp.float32)
    padi = jnp.zeros((RCH, 128 - NBLK), jnp.int32)
    bmax_ref[...] = jnp.concatenate([m, padf], axis=-1)
    barg_ref[...] = jnp.concatenate([il + row * BLK, padi], axis=-1)


def _tc_stage(rel_q, anch_q, scores_q):
    boxes = pl.pallas_call(
        _tc_decode_body,
        out_shape=jax.ShapeDtypeStruct((4, B, N), jnp.float32),
    )(rel_q, anch_q)
    probs, bmax, barg = pl.pallas_call(
        _tc_scores_body,
        grid=(B * CP // RCH,),
        in_specs=[pl.BlockSpec((RCH, N), lambda i: (i, 0))],
        out_specs=[
            pl.BlockSpec((RCH, N), lambda i: (i, 0)),
            pl.BlockSpec((RCH, 128), lambda i: (i, 0)),
            pl.BlockSpec((RCH, 128), lambda i: (i, 0)),
        ],
        out_shape=[
            jax.ShapeDtypeStruct((B * CP, N), jnp.float32),
            jax.ShapeDtypeStruct((B * CP, 128), jnp.float32),
            jax.ShapeDtypeStruct((B * CP, 128), jnp.int32),
        ],
    )(scores_q)
    return boxes, probs, bmax, barg


_IOTA = functools.partial(lax.iota, jnp.int32, LANES)
_BIG_I = 1 << 30


def _vex_f(ref, idx):
    base = (idx // LANES) * LANES
    v = ref[pl.ds(base, LANES)]
    return lax.reduce_sum(jnp.where(_IOTA() == idx - base, v, 0.0), (0,))


def _vex_i(ref, idx):
    base = (idx // LANES) * LANES
    v = ref[pl.ds(base, LANES)]
    return lax.reduce_sum(jnp.where(_IOTA() == idx - base, v, 0), (0,))


def _vbro(ref, idx):
    base = (idx // LANES) * LANES
    v = ref[pl.ds(base, LANES)]
    lanes = jnp.full((LANES,), idx - base, jnp.int32)
    return v.at[lanes].get(mode="promise_in_bounds")


def _vset(ref, idx, val):
    base = (idx // LANES) * LANES
    v = ref[pl.ds(base, LANES)]
    ref[pl.ds(base, LANES)] = jnp.where(_IOTA() == idx - base, val, v)


def _scan_range(ref, start, nslices, unroll=False):
    def step(i, carry):
        vmax, vidx = carry
        off = start + i * LANES
        v = ref[pl.ds(off, LANES)]
        take = v > vmax
        return (jnp.where(take, v, vmax),
                jnp.where(take, off + _IOTA(), vidx))
    carry = (jnp.full((LANES,), NEG, jnp.float32), jnp.zeros((LANES,), jnp.int32))
    if unroll:
        for i in range(nslices):
            carry = step(i, carry)
        vmax, vidx = carry
    else:
        vmax, vidx = lax.fori_loop(0, nslices, step, carry)
    m = lax.reduce_max(vmax, (0,))
    idx = lax.reduce_min(jnp.where(vmax == m, vidx, _BIG_I), (0,))
    return m, idx


def _find_global(bm, nslices):
    return _scan_range(bm, 0, nslices, unroll=True)


def _nms_class(pv, py1, px1, py2, px2, bm, ba, sy1, sx1, sy2, sx2, ssc):
    zeros16 = jnp.zeros((LANES,), jnp.float32)
    negs16 = jnp.full((LANES,), NEG, jnp.float32)

    def init_sel(i, _):
        off = pl.ds(i * LANES, LANES)
        sy1[off] = zeros16
        sx1[off] = zeros16
        sy2[off] = zeros16
        sx2[off] = zeros16
        ssc[off] = negs16
        return 0
    lax.fori_loop(0, KP // LANES, init_sel, 0)

    g0, gb0 = _find_global(bm, NBLKP // LANES)

    def nms_cond(carry):
        nsel, gmax, _ = carry
        return (nsel < K) & (gmax > 0.0)

    def nms_body(carry):
        nsel, gmax, gblk = carry
        idx = _vex_i(ba, gblk)
        by1 = _vbro(py1, idx)
        bx1 = _vbro(px1, idx)
        by2 = _vbro(py2, idx)
        bx2 = _vbro(px2, idx)
        a1 = (by2 - by1) * (bx2 - bx1)

        def iou_step(i, acc):
            off = pl.ds(i * LANES, LANES)
            vy1 = sy1[off]
            vx1 = sx1[off]
            vy2 = sy2[off]
            vx2 = sx2[off]
            yy1 = jnp.maximum(by1, vy1)
            xx1 = jnp.maximum(bx1, vx1)
            yy2 = jnp.minimum(by2, vy2)
            xx2 = jnp.minimum(bx2, vx2)
            inter = (jnp.maximum(yy2 - yy1, 0.0)
                     * jnp.maximum(xx2 - xx1, 0.0))
            a2 = (vy2 - vy1) * (vx2 - vx1)
            iou = inter / (a1 + a2 - inter + 1e-8)
            return acc | (iou > IOU_THR)
        supm = jnp.zeros((LANES,), jnp.bool_)
        for i in range(KP // LANES):
            supm = iou_step(i, supm)
        sup = jnp.any(supm)

        _vset(sy1, nsel, jnp.where(sup, 0.0, by1))
        _vset(sx1, nsel, jnp.where(sup, 0.0, bx1))
        _vset(sy2, nsel, jnp.where(sup, 0.0, by2))
        _vset(sx2, nsel, jnp.where(sup, 0.0, bx2))
        _vset(ssc, nsel, jnp.where(sup, NEG, gmax))
        nsel = nsel + jnp.where(sup, 0, 1)

        _vset(pv, idx, NEG)
        m, fidx = _scan_range(pv, gblk * BLK, BLK // LANES, unroll=True)
        _vset(bm, gblk, m)
        _vset(ba, gblk, fidx)
        gmax2, gblk2 = _find_global(bm, NBLKP // LANES)
        return nsel, gmax2, gblk2

    lax.while_loop(nms_cond, nms_body, (jnp.int32(0), g0, gb0))


def _merge_batch(mc_y1, mc_x1, mc_y2, mc_x2, mc_sc, mbm, mba,
                 st_bx, st_sc, st_lb, st_nv):
    zeros16 = jnp.zeros((LANES,), jnp.float32)

    def init_mblk(b, _):
        m, idx = _scan_range(mc_sc, b * MBLK, MBLK // LANES)
        _vset(mbm, b, m)
        _vset(mba, b, idx)
        return 0
    lax.fori_loop(0, MTOT // MBLK, init_mblk, 0)

    def pad_mblk(i, _):
        _vset(mbm, MTOT // MBLK + i, NEG)
        _vset(mba, MTOT // MBLK + i, 0)
        return 0
    lax.fori_loop(0, 2 * LANES - MTOT // MBLK, pad_mblk, 0)

    def sel_step(i, nv):
        s, blk = _find_global(mbm, 2)
        f = _vex_i(mba, blk)
        valid = s > NEG / 2.0
        cc = f // KP
        by1 = _vbro(mc_y1, f)
        bx1 = _vbro(mc_x1, f)
        by2 = _vbro(mc_y2, f)
        bx2 = _vbro(mc_x2, f)
        sw = jnp.where(valid, s, 0.0)
        lw = jnp.where(valid, cc.astype(jnp.float32), 0.0)
        _vset(st_sc, i, sw)
        _vset(st_lb, i, lw)
        base = (i * 4 // LANES) * LANES
        off = i * 4 - base
        io = _IOTA()
        v = st_bx[pl.ds(base, LANES)]
        v = jnp.where(io == off, jnp.where(valid, by1, 0.0), v)
        v = jnp.where(io == off + 1, jnp.where(valid, bx1, 0.0), v)
        v = jnp.where(io == off + 2, jnp.where(valid, by2, 0.0), v)
        v = jnp.where(io == off + 3, jnp.where(valid, bx2, 0.0), v)
        st_bx[pl.ds(base, LANES)] = v
        _vset(mc_sc, f, NEG)
        m, fidx = _scan_range(mc_sc, blk * MBLK, MBLK // LANES, unroll=True)
        _vset(mbm, blk, m)
        _vset(mba, blk, fidx)
        return nv + jnp.where(valid, 1, 0)
    nv = lax.fori_loop(0, K, sel_step, jnp.int32(0))

    def pad_out(i, _):
        _vset(st_sc, K + i, 0.0)
        _vset(st_lb, K + i, 0.0)
        return 0
    lax.fori_loop(0, KP - K, pad_out, 0)

    def pad_bx(i, _):
        st_bx[pl.ds(K * 4 + i * LANES, LANES)] = zeros16
        return 0
    lax.fori_loop(0, (KP - K) * 4 // LANES, pad_bx, 0)

    st_nv[pl.ds(0, LANES)] = jnp.where(_IOTA() == 0, nv, 0)


def _sc_body(probs_hbm, boxes_hbm, bmax_hbm, barg_hbm, oboxes, oscores, olabels, onv,
             py1, px1, py2, px2, pv, bm, ba,
             sy1, sx1, sy2, sx2, ssc,
             sh_y1, sh_x1, sh_y2, sh_x2, sh_sc,
             mc_y1, mc_x1, mc_y2, mc_x2, mc_sc,
             mbm, mba, st_bx, st_sc, st_lb, st_nv):
    cidx = lax.axis_index("c")
    sidx = lax.axis_index("s")
    batch = cidx * 2 + sidx // 8
    j = sidx % 8
    bb = sidx // 8

    pltpu.sync_copy(boxes_hbm.at[0, batch], py1)
    pltpu.sync_copy(boxes_hbm.at[1, batch], px1)
    pltpu.sync_copy(boxes_hbm.at[2, batch], py2)
    pltpu.sync_copy(boxes_hbm.at[3, batch], px2)

    zeros16 = jnp.zeros((LANES,), jnp.float32)
    negs16 = jnp.full((LANES,), NEG, jnp.float32)

    for t in range(3):
        c = j + 8 * t

        @pl.when(c < C)
        def _():
            pltpu.sync_copy(probs_hbm.at[batch * CP + c], pv)
            pltpu.sync_copy(bmax_hbm.at[batch * CP + c], bm)
            pltpu.sync_copy(barg_hbm.at[batch * CP + c], ba.at[pl.ds(0, 128)])
            _nms_class(pv, py1, px1, py2, px2, bm, ba, sy1, sx1, sy2, sx2, ssc)
            sh_off = bb * (C * KP + 16) + c * KP
            pltpu.sync_copy(sy1, sh_y1.at[pl.ds(sh_off, KP)])
            pltpu.sync_copy(sx1, sh_x1.at[pl.ds(sh_off, KP)])
            pltpu.sync_copy(sy2, sh_y2.at[pl.ds(sh_off, KP)])
            pltpu.sync_copy(sx2, sh_x2.at[pl.ds(sh_off, KP)])
            pltpu.sync_copy(ssc, sh_sc.at[pl.ds(sh_off, KP)])

    plsc.subcore_barrier()

    @pl.when(j == 5)
    def _():
        sh_b = bb * (C * KP + 16)
        pltpu.sync_copy(sh_y1.at[pl.ds(sh_b, C * KP + 16)], mc_y1)
        pltpu.sync_copy(sh_x1.at[pl.ds(sh_b, C * KP + 16)], mc_x1)
        pltpu.sync_copy(sh_y2.at[pl.ds(sh_b, C * KP + 16)], mc_y2)
        pltpu.sync_copy(sh_x2.at[pl.ds(sh_b, C * KP + 16)], mc_x2)
        pltpu.sync_copy(sh_sc.at[pl.ds(sh_b, C * KP)], mc_sc)

        _merge_batch(mc_y1, mc_x1, mc_y2, mc_x2, mc_sc, mbm, mba,
                     st_bx, st_sc, st_lb, st_nv)

        pltpu.sync_copy(st_bx, oboxes.at[batch])
        pltpu.sync_copy(st_sc, oscores.at[batch])
        pltpu.sync_copy(st_lb, olabels.at[batch])
        pltpu.sync_copy(st_nv, onv.at[batch])


def _sc_stage(probs, boxes_t, bmax, barg):
    mesh = plsc.VectorSubcoreMesh(core_axis_name="c", subcore_axis_name="s")
    f = pl.kernel(
        _sc_body,
        out_type=[
            jax.ShapeDtypeStruct((B, KP * 4), jnp.float32),
            jax.ShapeDtypeStruct((B, KP), jnp.float32),
            jax.ShapeDtypeStruct((B, KP), jnp.float32),
            jax.ShapeDtypeStruct((B, LANES), jnp.int32),
        ],
        mesh=mesh,
        compiler_params=pltpu.CompilerParams(needs_layout_passes=False),
        scratch_types=[
            pltpu.VMEM((N + 16,), jnp.float32),
            pltpu.VMEM((N + 16,), jnp.float32),
            pltpu.VMEM((N + 16,), jnp.float32),
            pltpu.VMEM((N + 16,), jnp.float32),
            pltpu.VMEM((N,), jnp.float32),
            pltpu.VMEM((128,), jnp.float32),
            pltpu.VMEM((128,), jnp.int32),
            pltpu.VMEM((KP,), jnp.float32),
            pltpu.VMEM((KP,), jnp.float32),
            pltpu.VMEM((KP,), jnp.float32),
            pltpu.VMEM((KP,), jnp.float32),
            pltpu.VMEM((KP,), jnp.float32),
            pltpu.VMEM_SHARED((2 * (C * KP + 16),), jnp.float32),
            pltpu.VMEM_SHARED((2 * (C * KP + 16),), jnp.float32),
            pltpu.VMEM_SHARED((2 * (C * KP + 16),), jnp.float32),
            pltpu.VMEM_SHARED((2 * (C * KP + 16),), jnp.float32),
            pltpu.VMEM_SHARED((2 * (C * KP + 16),), jnp.float32),
            pltpu.VMEM((MTOT + 16,), jnp.float32),
            pltpu.VMEM((MTOT + 16,), jnp.float32),
            pltpu.VMEM((MTOT + 16,), jnp.float32),
            pltpu.VMEM((MTOT + 16,), jnp.float32),
            pltpu.VMEM((MTOT,), jnp.float32),
            pltpu.VMEM((2 * LANES,), jnp.float32),
            pltpu.VMEM((3 * LANES,), jnp.int32),
            pltpu.VMEM((KP * 4,), jnp.float32),
            pltpu.VMEM((KP,), jnp.float32),
            pltpu.VMEM((KP,), jnp.float32),
            pltpu.VMEM((LANES,), jnp.int32),
        ],
    )
    return f(probs, boxes_t, bmax, barg)


def kernel(rel_codes, scores, anchors):
    rel_q = jnp.transpose(rel_codes, (2, 0, 1))
    anch_q = jnp.transpose(anchors, (1, 0))
    scores_p = jnp.pad(scores, ((0, 0), (0, 0), (0, CP - C)))
    scores_q = jnp.transpose(scores_p, (0, 2, 1)).reshape(B * CP, N)
    boxes3d, probs, bmax, barg = _tc_stage(rel_q, anch_q, scores_q)
    boxes_t = jnp.pad(boxes3d, ((0, 0), (0, 0), (0, 16)))
    bx, sc, lb, nv = _sc_stage(probs, boxes_t, bmax, barg)
    out_boxes = bx.reshape(B, KP, 4)[:, :K, :]
    out_scores = sc[:, :K]
    out_labels = lb[:, :K]
    num_valid = nv[:, 0]
    return out_boxes, out_scores, out_labels, num_valid


# speedup vs baseline: 1.5391x; 1.0037x over previous
import functools
import jax
import jax.numpy as jnp
from jax import lax
from jax.experimental import pallas as pl
from jax.experimental.pallas import tpu as pltpu
from jax.experimental.pallas import tpu_sc as plsc

B = 4
N = 20000
C = 21
NEG = -1e9
THR = 0.3
IOU_THR = 0.5
K = 100
KP = 112
BLK = 160
NBLK = 125
NBLKP = 128
RCH = 32
MBLK = 112
MTOT = 2352
LANES = 16
CP = 32


def _tc_decode_body(rel_ref, anch_ref, boxes_ref):
    ay1 = anch_ref[0]
    ax1 = anch_ref[1]
    ay2 = anch_ref[2]
    ax2 = anch_ref[3]
    ycenter_a = (ay1 + ay2) / 2.0
    xcenter_a = (ax1 + ax2) / 2.0
    ha = ay2 - ay1
    wa = ax2 - ax1
    ty = rel_ref[0] / 10.0
    tx = rel_ref[1] / 10.0
    th = rel_ref[2] / 5.0
    tw = rel_ref[3] / 5.0
    h = jnp.exp(th) * ha
    w = jnp.exp(tw) * wa
    yc = ty * ha + ycenter_a
    xc = tx * wa + xcenter_a
    boxes_ref[0] = jnp.clip(yc - h / 2.0, 0.0, 512.0)
    boxes_ref[1] = jnp.clip(xc - w / 2.0, 0.0, 512.0)
    boxes_ref[2] = jnp.clip(yc + h / 2.0, 0.0, 512.0)
    boxes_ref[3] = jnp.clip(xc + w / 2.0, 0.0, 512.0)


def _tc_scores_body(sc_ref, probs_ref, bmax_ref, barg_ref):
    p = 1.0 / (1.0 + jnp.exp(-sc_ref[...]))
    pt = jnp.where(p > THR, p, NEG)
    probs_ref[...] = pt
    pt3 = pt.reshape(RCH, NBLK, BLK)
    m = jnp.max(pt3, axis=-1)
    lane = lax.broadcasted_iota(jnp.int32, (RCH, NBLK, BLK), 2)
    il = jnp.min(jnp.where(pt3 == m[..., None], lane, _BIG_I), axis=-1)
    row = lax.broadcasted_iota(jnp.int32, (RCH, NBLK), 1)
    padf = jnp.full((RCH, 128 - NBLK), NEG, jnp.float32)
    padi = jnp.zeros((RCH, 128 - NBLK), jnp.int32)
    bmax_ref[...] = jnp.concatenate([m, padf], axis=-1)
    barg_ref[...] = jnp.concatenate([il + row * BLK, padi], axis=-1)


def _tc_stage(rel_q, anch_q, scores_q):
    boxes = pl.pallas_call(
        _tc_decode_body,
        out_shape=jax.ShapeDtypeStruct((4, B, N), jnp.float32),
    )(rel_q, anch_q)
    probs, bmax, barg = pl.pallas_call(
        _tc_scores_body,
        grid=(B * CP // RCH,),
        in_specs=[pl.BlockSpec((RCH, N), lambda i: (i, 0))],
        out_specs=[
            pl.BlockSpec((RCH, N), lambda i: (i, 0)),
            pl.BlockSpec((RCH, 128), lambda i: (i, 0)),
            pl.BlockSpec((RCH, 128), lambda i: (i, 0)),
        ],
        out_shape=[
            jax.ShapeDtypeStruct((B * CP, N), jnp.float32),
            jax.ShapeDtypeStruct((B * CP, 128), jnp.float32),
            jax.ShapeDtypeStruct((B * CP, 128), jnp.int32),
        ],
    )(scores_q)
    return boxes, probs, bmax, barg


_IOTA = functools.partial(lax.iota, jnp.int32, LANES)
_BIG_I = 1 << 30


def _vex_f(ref, idx):
    base = (idx // LANES) * LANES
    v = ref[pl.ds(base, LANES)]
    return lax.reduce_sum(jnp.where(_IOTA() == idx - base, v, 0.0), (0,))


def _vex_i(ref, idx):
    base = (idx // LANES) * LANES
    v = ref[pl.ds(base, LANES)]
    return lax.reduce_sum(jnp.where(_IOTA() == idx - base, v, 0), (0,))


def _vbro(ref, idx):
    base = (idx // LANES) * LANES
    v = ref[pl.ds(base, LANES)]
    lanes = jnp.full((LANES,), idx - base, jnp.int32)
    return v.at[lanes].get(mode="promise_in_bounds")


def _vset(ref, idx, val):
    base = (idx // LANES) * LANES
    v = ref[pl.ds(base, LANES)]
    ref[pl.ds(base, LANES)] = jnp.where(_IOTA() == idx - base, val, v)


def _scan_range(ref, start, nslices, unroll=False):
    def step(i, carry):
        vmax, vidx = carry
        off = start + i * LANES
        v = ref[pl.ds(off, LANES)]
        take = v > vmax
        return (jnp.where(take, v, vmax),
                jnp.where(take, off + _IOTA(), vidx))
    carry = (jnp.full((LANES,), NEG, jnp.float32), jnp.zeros((LANES,), jnp.int32))
    if unroll:
        for i in range(nslices):
            carry = step(i, carry)
        vmax, vidx = carry
    else:
        vmax, vidx = lax.fori_loop(0, nslices, step, carry)
    m = lax.reduce_max(vmax, (0,))
    idx = lax.reduce_min(jnp.where(vmax == m, vidx, _BIG_I), (0,))
    return m, idx


def _find_global(bm, nslices):
    return _scan_range(bm, 0, nslices, unroll=True)


def _nms_class(pv, py1, px1, py2, px2, bm, ba, sy1, sx1, sy2, sx2, ssc):
    zeros16 = jnp.zeros((LANES,), jnp.float32)
    negs16 = jnp.full((LANES,), NEG, jnp.float32)

    def init_sel(i, _):
        off = pl.ds(i * LANES, LANES)
        sy1[off] = zeros16
        sx1[off] = zeros16
        sy2[off] = zeros16
        sx2[off] = zeros16
        ssc[off] = negs16
        return 0
    lax.fori_loop(0, KP // LANES, init_sel, 0)

    g0, gb0 = _find_global(bm, NBLKP // LANES)

    def nms_cond(carry):
        nsel, gmax, _ = carry
        return (nsel < K) & (gmax > 0.0)

    def nms_body(carry):
        nsel, gmax, gblk = carry
        idx = _vex_i(ba, gblk)
        by1 = _vbro(py1, idx)
        bx1 = _vbro(px1, idx)
        by2 = _vbro(py2, idx)
        bx2 = _vbro(px2, idx)
        a1 = (by2 - by1) * (bx2 - bx1)

        def iou_step(i, acc):
            off = pl.ds(i * LANES, LANES)
            vy1 = sy1[off]
            vx1 = sx1[off]
            vy2 = sy2[off]
            vx2 = sx2[off]
            yy1 = jnp.maximum(by1, vy1)
            xx1 = jnp.maximum(bx1, vx1)
            yy2 = jnp.minimum(by2, vy2)
            xx2 = jnp.minimum(bx2, vx2)
            inter = (jnp.maximum(yy2 - yy1, 0.0)
                     * jnp.maximum(xx2 - xx1, 0.0))
            a2 = (vy2 - vy1) * (vx2 - vx1)
            iou = inter / (a1 + a2 - inter + 1e-8)
            return acc | (iou > IOU_THR)
        supm = jnp.zeros((LANES,), jnp.bool_)
        for i in range(KP // LANES):
            supm = iou_step(i, supm)
        sup = jnp.any(supm)

        _vset(sy1, nsel, jnp.where(sup, 0.0, by1))
        _vset(sx1, nsel, jnp.where(sup, 0.0, bx1))
        _vset(sy2, nsel, jnp.where(sup, 0.0, by2))
        _vset(sx2, nsel, jnp.where(sup, 0.0, bx2))
        _vset(ssc, nsel, jnp.where(sup, NEG, gmax))
        nsel = nsel + jnp.where(sup, 0, 1)

        _vset(pv, idx, NEG)
        m, fidx = _scan_range(pv, gblk * BLK, BLK // LANES, unroll=True)
        _vset(bm, gblk, m)
        _vset(ba, gblk, fidx)
        gmax2, gblk2 = _find_global(bm, NBLKP // LANES)
        return nsel, gmax2, gblk2

    lax.while_loop(nms_cond, nms_body, (jnp.int32(0), g0, gb0))


def _merge_batch(mc_y1, mc_x1, mc_y2, mc_x2, mc_sc, mbm, mba,
                 st_bx, st_sc, st_lb, st_nv):
    zeros16 = jnp.zeros((LANES,), jnp.float32)

    def init_mblk(b, _):
        m, idx = _scan_range(mc_sc, b * MBLK, MBLK // LANES)
        _vset(mbm, b, m)
        _vset(mba, b, idx)
        return 0
    lax.fori_loop(0, MTOT // MBLK, init_mblk, 0)

    def pad_mblk(i, _):
        _vset(mbm, MTOT // MBLK + i, NEG)
        _vset(mba, MTOT // MBLK + i, 0)
        return 0
    lax.fori_loop(0, 2 * LANES - MTOT // MBLK, pad_mblk, 0)

    def sel_step(i, nv):
        s, blk = _find_global(mbm, 2)
        f = _vex_i(mba, blk)
        valid = s > NEG / 2.0
        cc = f // KP
        by1 = _vbro(mc_y1, f)
        bx1 = _vbro(mc_x1, f)
        by2 = _vbro(mc_y2, f)
        bx2 = _vbro(mc_x2, f)
        sw = jnp.where(valid, s, 0.0)
        lw = jnp.where(valid, cc.astype(jnp.float32), 0.0)
        _vset(st_sc, i, sw)
        _vset(st_lb, i, lw)
        base = (i * 4 // LANES) * LANES
        off = i * 4 - base
        io = _IOTA()
        v = st_bx[pl.ds(base, LANES)]
        v = jnp.where(io == off, jnp.where(valid, by1, 0.0), v)
        v = jnp.where(io == off + 1, jnp.where(valid, bx1, 0.0), v)
        v = jnp.where(io == off + 2, jnp.where(valid, by2, 0.0), v)
        v = jnp.where(io == off + 3, jnp.where(valid, bx2, 0.0), v)
        st_bx[pl.ds(base, LANES)] = v
        _vset(mc_sc, f, NEG)
        m, fidx = _scan_range(mc_sc, blk * MBLK, MBLK // LANES, unroll=True)
        _vset(mbm, blk, m)
        _vset(mba, blk, fidx)
        return nv + jnp.where(valid, 1, 0)
    nv = lax.fori_loop(0, K, sel_step, jnp.int32(0))

    def pad_out(i, _):
        _vset(st_sc, K + i, 0.0)
        _vset(st_lb, K + i, 0.0)
        return 0
    lax.fori_loop(0, KP - K, pad_out, 0)

    def pad_bx(i, _):
        st_bx[pl.ds(K * 4 + i * LANES, LANES)] = zeros16
        return 0
    lax.fori_loop(0, (KP - K) * 4 // LANES, pad_bx, 0)

    st_nv[pl.ds(0, LANES)] = jnp.where(_IOTA() == 0, nv, 0)


def _sc_body(probs_hbm, boxes_hbm, bmax_hbm, barg_hbm, oboxes, oscores, olabels, onv,
             py1, px1, py2, px2, pv, bm, ba,
             sy1, sx1, sy2, sx2, ssc,
             sh_y1, sh_x1, sh_y2, sh_x2, sh_sc,
             mc_y1, mc_x1, mc_y2, mc_x2, mc_sc,
             mbm, mba, st_bx, st_sc, st_lb, st_nv):
    cidx = lax.axis_index("c")
    sidx = lax.axis_index("s")
    batch = cidx * 2 + sidx // 8
    j = sidx % 8
    bb = sidx // 8

    pltpu.sync_copy(boxes_hbm.at[0, batch], py1)
    pltpu.sync_copy(boxes_hbm.at[1, batch], px1)
    pltpu.sync_copy(boxes_hbm.at[2, batch], py2)
    pltpu.sync_copy(boxes_hbm.at[3, batch], px2)

    zeros16 = jnp.zeros((LANES,), jnp.float32)
    negs16 = jnp.full((LANES,), NEG, jnp.float32)

    for t in range(3):
        c = j + 8 * t

        @pl.when(c < C)
        def _():
            pltpu.sync_copy(probs_hbm.at[batch * CP + c], pv)
            pltpu.sync_copy(bmax_hbm.at[batch * CP + c], bm)
            pltpu.sync_copy(barg_hbm.at[batch * CP + c], ba.at[pl.ds(0, 128)])
            _nms_class(pv, py1, px1, py2, px2, bm, ba, sy1, sx1, sy2, sx2, ssc)
            sh_off = bb * (C * KP + 16) + c * KP
            pltpu.sync_copy(sy1, sh_y1.at[pl.ds(sh_off, KP)])
            pltpu.sync_copy(sx1, sh_x1.at[pl.ds(sh_off, KP)])
            pltpu.sync_copy(sy2, sh_y2.at[pl.ds(sh_off, KP)])
            pltpu.sync_copy(sx2, sh_x2.at[pl.ds(sh_off, KP)])
            pltpu.sync_copy(ssc, sh_sc.at[pl.ds(sh_off, KP)])

    plsc.subcore_barrier()

    @pl.when(j == 5)
    def _():
        sh_b = bb * (C * KP + 16)
        pltpu.sync_copy(sh_y1.at[pl.ds(sh_b, C * KP + 16)], mc_y1)
        pltpu.sync_copy(sh_x1.at[pl.ds(sh_b, C * KP + 16)], mc_x1)
        pltpu.sync_copy(sh_y2.at[pl.ds(sh_b, C * KP + 16)], mc_y2)
        pltpu.sync_copy(sh_x2.at[pl.ds(sh_b, C * KP + 16)], mc_x2)
        pltpu.sync_copy(sh_sc.at[pl.ds(sh_b, C * KP)], mc_sc)

        _merge_batch(mc_y1, mc_x1, mc_y2, mc_x2, mc_sc, mbm, mba,
                     st_bx, st_sc, st_lb, st_nv)

        pltpu.sync_copy(st_bx, oboxes.at[batch])
        pltpu.sync_copy(st_sc, oscores.at[batch])
        pltpu.sync_copy(st_lb, olabels.at[batch])
        pltpu.sync_copy(st_nv, onv.at[batch])


def _sc_stage(probs, boxes_t, bmax, barg):
    mesh = plsc.VectorSubcoreMesh(core_axis_name="c", subcore_axis_name="s")
    f = pl.kernel(
        _sc_body,
        out_type=[
            jax.ShapeDtypeStruct((B, KP * 4), jnp.float32),
            jax.ShapeDtypeStruct((B, KP), jnp.float32),
            jax.ShapeDtypeStruct((B, KP), jnp.float32),
            jax.ShapeDtypeStruct((B, LANES), jnp.int32),
        ],
        mesh=mesh,
        compiler_params=pltpu.CompilerParams(needs_layout_passes=False),
        scratch_types=[
            pltpu.VMEM((N + 16,), jnp.float32),
            pltpu.VMEM((N + 16,), jnp.float32),
            pltpu.VMEM((N + 16,), jnp.float32),
            pltpu.VMEM((N + 16,), jnp.float32),
            pltpu.VMEM((N,), jnp.float32),
            pltpu.VMEM((128,), jnp.float32),
            pltpu.VMEM((128,), jnp.int32),
            pltpu.VMEM((KP,), jnp.float32),
            pltpu.VMEM((KP,), jnp.float32),
            pltpu.VMEM((KP,), jnp.float32),
            pltpu.VMEM((KP,), jnp.float32),
            pltpu.VMEM((KP,), jnp.float32),
            pltpu.VMEM_SHARED((2 * (C * KP + 16),), jnp.float32),
            pltpu.VMEM_SHARED((2 * (C * KP + 16),), jnp.float32),
            pltpu.VMEM_SHARED((2 * (C * KP + 16),), jnp.float32),
            pltpu.VMEM_SHARED((2 * (C * KP + 16),), jnp.float32),
            pltpu.VMEM_SHARED((2 * (C * KP + 16),), jnp.float32),
            pltpu.VMEM((MTOT + 16,), jnp.float32),
            pltpu.VMEM((MTOT + 16,), jnp.float32),
            pltpu.VMEM((MTOT + 16,), jnp.float32),
            pltpu.VMEM((MTOT + 16,), jnp.float32),
            pltpu.VMEM((MTOT,), jnp.float32),
            pltpu.VMEM((2 * LANES,), jnp.float32),
            pltpu.VMEM((3 * LANES,), jnp.int32),
            pltpu.VMEM((KP * 4,), jnp.float32),
            pltpu.VMEM((KP,), jnp.float32),
            pltpu.VMEM((KP,), jnp.float32),
            pltpu.VMEM((LANES,), jnp.int32),
        ],
    )
    return f(probs, boxes_t, bmax, barg)


def kernel(rel_codes, scores, anchors):
    rel_q = jnp.transpose(rel_codes, (2, 0, 1))
    anch_q = jnp.transpose(anchors, (1, 0))
    scores_p = jnp.pad(scores, ((0, 0), (0, 0), (0, CP - C)))
    scores_q = jnp.transpose(scores_p, (0, 2, 1)).reshape(B * CP, N)
    boxes3d, probs, bmax, barg = _tc_stage(rel_q, anch_q, scores_q)
    boxes_t = jnp.pad(boxes3d, ((0, 0), (0, 0), (0, 16)))
    bx, sc, lb, nv = _sc_stage(probs, boxes_t, bmax, barg)
    out_boxes = bx.reshape(B, KP, 4)[:, :K, :]
    out_scores = sc[:, :K]
    out_labels = lb[:, :K]
    num_valid = nv[:, 0]
    return out_boxes, out_scores, out_labels, num_valid


# speedup vs baseline: 1.5407x; 1.0010x over previous
import functools
import jax
import jax.numpy as jnp
from jax import lax
from jax.experimental import pallas as pl
from jax.experimental.pallas import tpu as pltpu
from jax.experimental.pallas import tpu_sc as plsc

B = 4
N = 20000
C = 21
NEG = -1e9
THR = 0.3
IOU_THR = 0.5
K = 100
KP = 112
BLK = 160
NBLK = 125
NBLKP = 128
RCH = 32
MBLK = 112
MTOT = 2352
LANES = 16
CP = 32


def _tc_decode_body(rel_ref, anch_ref, boxes_ref):
    ay1 = anch_ref[0]
    ax1 = anch_ref[1]
    ay2 = anch_ref[2]
    ax2 = anch_ref[3]
    ycenter_a = (ay1 + ay2) / 2.0
    xcenter_a = (ax1 + ax2) / 2.0
    ha = ay2 - ay1
    wa = ax2 - ax1
    ty = rel_ref[0] / 10.0
    tx = rel_ref[1] / 10.0
    th = rel_ref[2] / 5.0
    tw = rel_ref[3] / 5.0
    h = jnp.exp(th) * ha
    w = jnp.exp(tw) * wa
    yc = ty * ha + ycenter_a
    xc = tx * wa + xcenter_a
    boxes_ref[0] = jnp.clip(yc - h / 2.0, 0.0, 512.0)
    boxes_ref[1] = jnp.clip(xc - w / 2.0, 0.0, 512.0)
    boxes_ref[2] = jnp.clip(yc + h / 2.0, 0.0, 512.0)
    boxes_ref[3] = jnp.clip(xc + w / 2.0, 0.0, 512.0)


def _tc_scores_body(sc_ref, probs_ref, bmax_ref, barg_ref):
    p = 1.0 / (1.0 + jnp.exp(-sc_ref[...]))
    pt = jnp.where(p > THR, p, NEG)
    probs_ref[...] = pt
    pt3 = pt.reshape(RCH, NBLK, BLK)
    m = jnp.max(pt3, axis=-1)
    lane = lax.broadcasted_iota(jnp.int32, (RCH, NBLK, BLK), 2)
    il = jnp.min(jnp.where(pt3 == m[..., None], lane, _BIG_I), axis=-1)
    row = lax.broadcasted_iota(jnp.int32, (RCH, NBLK), 1)
    padf = jnp.full((RCH, 128 - NBLK), NEG, jnp.float32)
    padi = jnp.zeros((RCH, 128 - NBLK), jnp.int32)
    bmax_ref[...] = jnp.concatenate([m, padf], axis=-1)
    barg_ref[...] = jnp.concatenate([il + row * BLK, padi], axis=-1)


def _tc_stage(rel_q, anch_q, scores_q):
    boxes = pl.pallas_call(
        _tc_decode_body,
        out_shape=jax.ShapeDtypeStruct((4, B, N), jnp.float32),
    )(rel_q, anch_q)
    probs, bmax, barg = pl.pallas_call(
        _tc_scores_body,
        grid=(B * CP // RCH,),
        in_specs=[pl.BlockSpec((RCH, N), lambda i: (i, 0))],
        out_specs=[
            pl.BlockSpec((RCH, N), lambda i: (i, 0)),
            pl.BlockSpec((RCH, 128), lambda i: (i, 0)),
            pl.BlockSpec((RCH, 128), lambda i: (i, 0)),
        ],
        out_shape=[
            jax.ShapeDtypeStruct((B * CP, N), jnp.float32),
            jax.ShapeDtypeStruct((B * CP, 128), jnp.float32),
            jax.ShapeDtypeStruct((B * CP, 128), jnp.int32),
        ],
    )(scores_q)
    return boxes, probs, bmax, barg


_IOTA = functools.partial(lax.iota, jnp.int32, LANES)
_BIG_I = 1 << 30


def _vex_i(ref, idx):
    base = (idx // LANES) * LANES
    v = ref[pl.ds(base, LANES)]
    return lax.reduce_sum(jnp.where(_IOTA() == idx - base, v, 0), (0,))


def _vbro(ref, idx):
    base = (idx // LANES) * LANES
    v = ref[pl.ds(base, LANES)]
    lanes = jnp.full((LANES,), idx - base, jnp.int32)
    return v.at[lanes].get(mode="promise_in_bounds")


def _vset(ref, idx, val):
    base = (idx // LANES) * LANES
    v = ref[pl.ds(base, LANES)]
    ref[pl.ds(base, LANES)] = jnp.where(_IOTA() == idx - base, val, v)


def _scan_range(ref, start, nslices, unroll=False):
    def step(i, carry):
        vmax, vidx = carry
        off = start + i * LANES
        v = ref[pl.ds(off, LANES)]
        take = v > vmax
        return (jnp.where(take, v, vmax),
                jnp.where(take, off + _IOTA(), vidx))
    carry = (jnp.full((LANES,), NEG, jnp.float32), jnp.zeros((LANES,), jnp.int32))
    if unroll:
        for i in range(nslices):
            carry = step(i, carry)
        vmax, vidx = carry
    else:
        vmax, vidx = lax.fori_loop(0, nslices, step, carry)
    m = lax.reduce_max(vmax, (0,))
    idx = lax.reduce_min(jnp.where(vmax == m, vidx, _BIG_I), (0,))
    return m, idx


def _find_global(bm, nslices):
    return _scan_range(bm, 0, nslices, unroll=True)


def _nms_class(pv, py1, px1, py2, px2, bm, ba, sy1, sx1, sy2, sx2, ssc):
    zeros16 = jnp.zeros((LANES,), jnp.float32)
    negs16 = jnp.full((LANES,), NEG, jnp.float32)

    def init_sel(i, _):
        off = pl.ds(i * LANES, LANES)
        sy1[off] = zeros16
        sx1[off] = zeros16
        sy2[off] = zeros16
        sx2[off] = zeros16
        ssc[off] = negs16
        return 0
    lax.fori_loop(0, KP // LANES, init_sel, 0)

    g0, gb0 = _find_global(bm, NBLKP // LANES)

    def nms_cond(carry):
        nsel, gmax, _ = carry
        return (nsel < K) & (gmax > 0.0)

    def nms_body(carry):
        nsel, gmax, gblk = carry
        idx = _vex_i(ba, gblk)
        by1 = _vbro(py1, idx)
        bx1 = _vbro(px1, idx)
        by2 = _vbro(py2, idx)
        bx2 = _vbro(px2, idx)
        a1 = (by2 - by1) * (bx2 - bx1)

        def iou_step(i, acc):
            off = pl.ds(i * LANES, LANES)
            vy1 = sy1[off]
            vx1 = sx1[off]
            vy2 = sy2[off]
            vx2 = sx2[off]
            yy1 = jnp.maximum(by1, vy1)
            xx1 = jnp.maximum(bx1, vx1)
            yy2 = jnp.minimum(by2, vy2)
            xx2 = jnp.minimum(bx2, vx2)
            inter = (jnp.maximum(yy2 - yy1, 0.0)
                     * jnp.maximum(xx2 - xx1, 0.0))
            a2 = (vy2 - vy1) * (vx2 - vx1)
            iou = inter / (a1 + a2 - inter + 1e-8)
            return acc | (iou > IOU_THR)
        supm = jnp.zeros((LANES,), jnp.bool_)
        for i in range(KP // LANES):
            supm = iou_step(i, supm)
        sup = jnp.any(supm)

        _vset(sy1, nsel, jnp.where(sup, 0.0, by1))
        _vset(sx1, nsel, jnp.where(sup, 0.0, bx1))
        _vset(sy2, nsel, jnp.where(sup, 0.0, by2))
        _vset(sx2, nsel, jnp.where(sup, 0.0, bx2))
        _vset(ssc, nsel, jnp.where(sup, NEG, gmax))
        nsel = nsel + jnp.where(sup, 0, 1)

        _vset(pv, idx, NEG)
        m, fidx = _scan_range(pv, gblk * BLK, BLK // LANES, unroll=True)
        _vset(bm, gblk, m)
        _vset(ba, gblk, fidx)
        gmax2, gblk2 = _find_global(bm, NBLKP // LANES)
        return nsel, gmax2, gblk2

    lax.while_loop(nms_cond, nms_body, (jnp.int32(0), g0, gb0))


def _merge_batch(mc_y1, mc_x1, mc_y2, mc_x2, mc_sc, mbm, mba,
                 st_bx, st_sc, st_lb, st_nv):
    zeros16 = jnp.zeros((LANES,), jnp.float32)

    def init_mblk(b, _):
        m, idx = _scan_range(mc_sc, b * MBLK, MBLK // LANES)
        _vset(mbm, b, m)
        _vset(mba, b, idx)
        return 0
    lax.fori_loop(0, MTOT // MBLK, init_mblk, 0)

    def pad_mblk(i, _):
        _vset(mbm, MTOT // MBLK + i, NEG)
        _vset(mba, MTOT // MBLK + i, 0)
        return 0
    lax.fori_loop(0, 2 * LANES - MTOT // MBLK, pad_mblk, 0)

    def sel_step(i, nv):
        s, blk = _find_global(mbm, 2)
        f = _vex_i(mba, blk)
        valid = s > NEG / 2.0
        cc = f // KP
        by1 = _vbro(mc_y1, f)
        bx1 = _vbro(mc_x1, f)
        by2 = _vbro(mc_y2, f)
        bx2 = _vbro(mc_x2, f)
        sw = jnp.where(valid, s, 0.0)
        lw = jnp.where(valid, cc.astype(jnp.float32), 0.0)
        _vset(st_sc, i, sw)
        _vset(st_lb, i, lw)
        base = (i * 4 // LANES) * LANES
        off = i * 4 - base
        io = _IOTA()
        v = st_bx[pl.ds(base, LANES)]
        v = jnp.where(io == off, jnp.where(valid, by1, 0.0), v)
        v = jnp.where(io == off + 1, jnp.where(valid, bx1, 0.0), v)
        v = jnp.where(io == off + 2, jnp.where(valid, by2, 0.0), v)
        v = jnp.where(io == off + 3, jnp.where(valid, bx2, 0.0), v)
        st_bx[pl.ds(base, LANES)] = v
        _vset(mc_sc, f, NEG)
        m, fidx = _scan_range(mc_sc, blk * MBLK, MBLK // LANES, unroll=True)
        _vset(mbm, blk, m)
        _vset(mba, blk, fidx)
        return nv + jnp.where(valid, 1, 0)
    nv = lax.fori_loop(0, K, sel_step, jnp.int32(0))

    def pad_out(i, _):
        _vset(st_sc, K + i, 0.0)
        _vset(st_lb, K + i, 0.0)
        return 0
    lax.fori_loop(0, KP - K, pad_out, 0)

    def pad_bx(i, _):
        st_bx[pl.ds(K * 4 + i * LANES, LANES)] = zeros16
        return 0
    lax.fori_loop(0, (KP - K) * 4 // LANES, pad_bx, 0)

    st_nv[pl.ds(0, LANES)] = jnp.where(_IOTA() == 0, nv, 0)


def _sc_body(probs_hbm, boxes_hbm, bmax_hbm, barg_hbm, oboxes, oscores, olabels, onv,
             py1, px1, py2, px2, pv, bm, ba,
             sy1, sx1, sy2, sx2, ssc,
             sh_y1, sh_x1, sh_y2, sh_x2, sh_sc,
             mc_y1, mc_x1, mc_y2, mc_x2, mc_sc,
             mbm, mba, st_bx, st_sc, st_lb, st_nv):
    cidx = lax.axis_index("c")
    sidx = lax.axis_index("s")
    batch = cidx * 2 + sidx // 8
    j = sidx % 8
    bb = sidx // 8

    pltpu.sync_copy(boxes_hbm.at[0, batch], py1)
    pltpu.sync_copy(boxes_hbm.at[1, batch], px1)
    pltpu.sync_copy(boxes_hbm.at[2, batch], py2)
    pltpu.sync_copy(boxes_hbm.at[3, batch], px2)

    zeros16 = jnp.zeros((LANES,), jnp.float32)
    negs16 = jnp.full((LANES,), NEG, jnp.float32)

    for t in range(3):
        c = j + 8 * t

        @pl.when(c < C)
        def _():
            pltpu.sync_copy(probs_hbm.at[batch * CP + c], pv)
            pltpu.sync_copy(bmax_hbm.at[batch * CP + c], bm)
            pltpu.sync_copy(barg_hbm.at[batch * CP + c], ba.at[pl.ds(0, 128)])
            _nms_class(pv, py1, px1, py2, px2, bm, ba, sy1, sx1, sy2, sx2, ssc)
            sh_off = bb * (C * KP + 16) + c * KP
            pltpu.sync_copy(sy1, sh_y1.at[pl.ds(sh_off, KP)])
            pltpu.sync_copy(sx1, sh_x1.at[pl.ds(sh_off, KP)])
            pltpu.sync_copy(sy2, sh_y2.at[pl.ds(sh_off, KP)])
            pltpu.sync_copy(sx2, sh_x2.at[pl.ds(sh_off, KP)])
            pltpu.sync_copy(ssc, sh_sc.at[pl.ds(sh_off, KP)])

    plsc.subcore_barrier()

    @pl.when(j == 5)
    def _():
        sh_b = bb * (C * KP + 16)
        pltpu.sync_copy(sh_y1.at[pl.ds(sh_b, C * KP + 16)], mc_y1)
        pltpu.sync_copy(sh_x1.at[pl.ds(sh_b, C * KP + 16)], mc_x1)
        pltpu.sync_copy(sh_y2.at[pl.ds(sh_b, C * KP + 16)], mc_y2)
        pltpu.sync_copy(sh_x2.at[pl.ds(sh_b, C * KP + 16)], mc_x2)
        pltpu.sync_copy(sh_sc.at[pl.ds(sh_b, C * KP)], mc_sc)

        _merge_batch(mc_y1, mc_x1, mc_y2, mc_x2, mc_sc, mbm, mba,
                     st_bx, st_sc, st_lb, st_nv)

        pltpu.sync_copy(st_bx, oboxes.at[batch])
        pltpu.sync_copy(st_sc, oscores.at[batch])
        pltpu.sync_copy(st_lb, olabels.at[batch])
        pltpu.sync_copy(st_nv, onv.at[batch])


def _sc_stage(probs, boxes_t, bmax, barg):
    mesh = plsc.VectorSubcoreMesh(core_axis_name="c", subcore_axis_name="s")
    f = pl.kernel(
        _sc_body,
        out_type=[
            jax.ShapeDtypeStruct((B, KP * 4), jnp.float32),
            jax.ShapeDtypeStruct((B, KP), jnp.float32),
            jax.ShapeDtypeStruct((B, KP), jnp.float32),
            jax.ShapeDtypeStruct((B, LANES), jnp.int32),
        ],
        mesh=mesh,
        compiler_params=pltpu.CompilerParams(needs_layout_passes=False),
        scratch_types=[
            pltpu.VMEM((N + 16,), jnp.float32),
            pltpu.VMEM((N + 16,), jnp.float32),
            pltpu.VMEM((N + 16,), jnp.float32),
            pltpu.VMEM((N + 16,), jnp.float32),
            pltpu.VMEM((N,), jnp.float32),
            pltpu.VMEM((128,), jnp.float32),
            pltpu.VMEM((128,), jnp.int32),
            pltpu.VMEM((KP,), jnp.float32),
            pltpu.VMEM((KP,), jnp.float32),
            pltpu.VMEM((KP,), jnp.float32),
            pltpu.VMEM((KP,), jnp.float32),
            pltpu.VMEM((KP,), jnp.float32),
            pltpu.VMEM_SHARED((2 * (C * KP + 16),), jnp.float32),
            pltpu.VMEM_SHARED((2 * (C * KP + 16),), jnp.float32),
            pltpu.VMEM_SHARED((2 * (C * KP + 16),), jnp.float32),
            pltpu.VMEM_SHARED((2 * (C * KP + 16),), jnp.float32),
            pltpu.VMEM_SHARED((2 * (C * KP + 16),), jnp.float32),
            pltpu.VMEM((MTOT + 16,), jnp.float32),
            pltpu.VMEM((MTOT + 16,), jnp.float32),
            pltpu.VMEM((MTOT + 16,), jnp.float32),
            pltpu.VMEM((MTOT + 16,), jnp.float32),
            pltpu.VMEM((MTOT,), jnp.float32),
            pltpu.VMEM((2 * LANES,), jnp.float32),
            pltpu.VMEM((3 * LANES,), jnp.int32),
            pltpu.VMEM((KP * 4,), jnp.float32),
            pltpu.VMEM((KP,), jnp.float32),
            pltpu.VMEM((KP,), jnp.float32),
            pltpu.VMEM((LANES,), jnp.int32),
        ],
    )
    return f(probs, boxes_t, bmax, barg)


def kernel(rel_codes, scores, anchors):
    rel_q = jnp.transpose(rel_codes, (2, 0, 1))
    anch_q = jnp.transpose(anchors, (1, 0))
    scores_p = jnp.pad(scores, ((0, 0), (0, 0), (0, CP - C)))
    scores_q = jnp.transpose(scores_p, (0, 2, 1)).reshape(B * CP, N)
    boxes3d, probs, bmax, barg = _tc_stage(rel_q, anch_q, scores_q)
    boxes_t = jnp.pad(boxes3d, ((0, 0), (0, 0), (0, 16)))
    bx, sc, lb, nv = _sc_stage(probs, boxes_t, bmax, barg)
    out_boxes = bx.reshape(B, KP, 4)[:, :K, :]
    out_scores = sc[:, :K]
    out_labels = lb[:, :K]
    num_valid = nv[:, 0]
    return out_boxes, out_scores, out_labels, num_valid
